# baseline (speedup 1.0000x reference)
import functools

import jax
import jax.numpy as jnp
from jax import lax
from jax.experimental import pallas as pl
from jax.experimental.pallas import tpu as pltpu
from jax.experimental.pallas import tpu_sc as plsc

N_ITEM = 40000
N_FEAT = 10000
E = 400000
BS = 4096
EMB = 32
H1 = 8
OUT = 32

NC = 2
NS = 16
NW = NC * NS
SUB = 128
NSUB = 8
CHUNK = SUB * NSUB
E_PAD = ((E + CHUNK - 1) // CHUNK) * CHUNK
NCHUNKS = E_PAD // CHUNK

_mesh = plsc.VectorSubcoreMesh(core_axis_name="c", subcore_axis_name="s")
_sc_params = pltpu.CompilerParams(use_tc_tiling_on_sc=False,
                                  needs_layout_passes=False)


def _iota16():
    return lax.iota(jnp.int32, 16)


def _splat(x):
    return jnp.full((16,), x, jnp.int32)


def _vread(ref, i, off):
    return plsc.load_gather(ref, [_splat(i), _iota16() + off])


def _vwrite(ref, i, off, val):
    plsc.store_scatter(ref, [_splat(i), _iota16() + off], val)


def _edge_mask(gid):
    one = jnp.full((16,), 1.0, jnp.float32)
    zero = jnp.full((16,), 0.0, jnp.float32)
    return jnp.where(_splat(gid) < E, one, zero)


def _make_gather(n_rows_pad, table_rows):
    nloc = n_rows_pad // (NW * SUB)

    @functools.partial(
        pl.kernel, mesh=_mesh, compiler_params=_sc_params,
        out_type=jax.ShapeDtypeStruct((n_rows_pad, 32), jnp.float32),
        scratch_types=[
            pltpu.VMEM((SUB,), jnp.int32),
            pltpu.VMEM((SUB, 32), jnp.float32),
            pltpu.SemaphoreType.DMA,
        ],
    )
    def k(table_hbm, idx_hbm, out_hbm, idx_v, rows_v, sem):
        w = lax.axis_index("c") * NS + lax.axis_index("s")

        def body(j, carry):
            r = w * nloc + j
            pltpu.sync_copy(idx_hbm.at[r], idx_v)
            pltpu.async_copy(table_hbm.at[idx_v], rows_v, sem).wait()
            pltpu.sync_copy(rows_v, out_hbm.at[pl.ds(r * SUB, SUB)])
            return carry

        lax.fori_loop(0, nloc, body, 0)

    return k


def _make_kA(n_dst):
    rpt = n_dst // NS

    @functools.partial(
        pl.kernel, mesh=_mesh, compiler_params=_sc_params,
        out_type=jax.ShapeDtypeStruct((NC * n_dst, 16), jnp.float32),
        scratch_types=[
            pltpu.VMEM((NSUB, SUB), jnp.int32),
            pltpu.VMEM((NSUB, SUB), jnp.int32),
            pltpu.VMEM((CHUNK, 16), jnp.float32),
            pltpu.VMEM((CHUNK, 16), jnp.float32),
            pltpu.VMEM((CHUNK, 16), jnp.float32),
            pltpu.VMEM((16,), jnp.float32),
            pltpu.VMEM_SHARED((n_dst, 16), jnp.float32),
            pltpu.SemaphoreType.DMA,
            pltpu.SemaphoreType.DMA,
            pltpu.SemaphoreType.DMA,
        ],
    )
    def k(src_hbm, dst_hbm, el_hbm, er_hbm, c_hbm, zeros_hbm, den_out,
          src_v, dst_v, el_v, er_v, ee_v, c_v, acc, s_el, s_er, s_sc):
        cid = lax.axis_index("c")
        sid = lax.axis_index("s")
        w = cid * NS + sid
        pltpu.sync_copy(zeros_hbm.at[pl.ds(0, rpt)], acc.at[pl.ds(sid * rpt, rpt)])
        pltpu.sync_copy(c_hbm, c_v)
        plsc.subcore_barrier()
        nch = (NCHUNKS - 1 - w) // NW + 1

        def chunk(j, carry):
            g = w + j * NW
            pltpu.sync_copy(src_hbm.at[pl.ds(g * NSUB, NSUB)], src_v)
            pltpu.sync_copy(dst_hbm.at[pl.ds(g * NSUB, NSUB)], dst_v)
            cps = []
            for sb in range(NSUB):
                cps.append(pltpu.async_copy(
                    el_hbm.at[src_v.at[sb]],
                    el_v.at[pl.ds(sb * SUB, SUB)], s_el))
                cps.append(pltpu.async_copy(
                    er_hbm.at[dst_v.at[sb]],
                    er_v.at[pl.ds(sb * SUB, SUB)], s_er))
            for cp in cps:
                cp.wait()
            cvec = c_v[...]

            def edge(i, c2):
                ev = _vread(el_v, i, 0) + _vread(er_v, i, 0)
                ev = jnp.maximum(ev, 0.2 * ev)
                ee = jnp.exp(ev - cvec) * _edge_mask(g * CHUNK + i)
                _vwrite(ee_v, i, 0, ee)
                return c2

            lax.fori_loop(0, CHUNK, edge, 0, unroll=2)
            scs = []
            for sb in range(NSUB):
                scs.append(pltpu.async_copy(
                    ee_v.at[pl.ds(sb * SUB, SUB)],
                    acc.at[dst_v.at[sb]], s_sc, add=True))
            for cp in scs:
                cp.wait()
            return carry

        lax.fori_loop(0, nch, chunk, 0)
        plsc.subcore_barrier()
        pltpu.sync_copy(acc.at[pl.ds(sid * rpt, rpt)],
                        den_out.at[pl.ds(cid * n_dst + sid * rpt, rpt)])

    return k


def _make_kB():
    @functools.partial(
        pl.kernel, mesh=_mesh, compiler_params=_sc_params,
        out_type=jax.ShapeDtypeStruct((E_PAD, 16), jnp.float32),
        scratch_types=[
            pltpu.VMEM((NSUB, SUB), jnp.int32),
            pltpu.VMEM((NSUB, SUB), jnp.int32),
            pltpu.VMEM((CHUNK, 16), jnp.float32),
            pltpu.VMEM((CHUNK, 16), jnp.float32),
            pltpu.VMEM((CHUNK, 16), jnp.float32),
            pltpu.VMEM((CHUNK, 16), jnp.float32),
            pltpu.VMEM((CHUNK, 16), jnp.float32),
            pltpu.VMEM((16,), jnp.float32),
            pltpu.SemaphoreType.DMA,
            pltpu.SemaphoreType.DMA,
            pltpu.SemaphoreType.DMA,
            pltpu.SemaphoreType.DMA,
        ],
    )
    def k(src_hbm, dst_hbm, el_hbm, er_hbm, d0_hbm, d1_hbm, c_hbm, alpha_out,
          src_v, dst_v, el_v, er_v, d0_v, d1_v, a_v, c_v, s1, s2, s3, s4):
        cid = lax.axis_index("c")
        sid = lax.axis_index("s")
        w = cid * NS + sid
        pltpu.sync_copy(c_hbm, c_v)
        nch = (NCHUNKS - 1 - w) // NW + 1
        eps = jnp.full((16,), 1e-16, jnp.float32)

        def chunk(j, carry):
            g = w + j * NW
            pltpu.sync_copy(src_hbm.at[pl.ds(g * NSUB, NSUB)], src_v)
            pltpu.sync_copy(dst_hbm.at[pl.ds(g * NSUB, NSUB)], dst_v)
            cps = []
            for sb in range(NSUB):
                cps.append(pltpu.async_copy(
                    el_hbm.at[src_v.at[sb]],
                    el_v.at[pl.ds(sb * SUB, SUB)], s1))
                cps.append(pltpu.async_copy(
                    er_hbm.at[dst_v.at[sb]],
                    er_v.at[pl.ds(sb * SUB, SUB)], s2))
                cps.append(pltpu.async_copy(
                    d0_hbm.at[dst_v.at[sb]],
                    d0_v.at[pl.ds(sb * SUB, SUB)], s3))
                cps.append(pltpu.async_copy(
                    d1_hbm.at[dst_v.at[sb]],
                    d1_v.at[pl.ds(sb * SUB, SUB)], s4))
            for cp in cps:
                cp.wait()
            cvec = c_v[...]

            def edge(i, c2):
                ev = _vread(el_v, i, 0) + _vread(er_v, i, 0)
                ev = jnp.maximum(ev, 0.2 * ev)
                ee = jnp.exp(ev - cvec)
                den = _vread(d0_v, i, 0) + _vread(d1_v, i, 0)
                alpha = ee / jnp.maximum(den, eps)
                _vwrite(a_v, i, 0, alpha)
                return c2

            lax.fori_loop(0, CHUNK, edge, 0, unroll=2)
            pltpu.sync_copy(a_v, alpha_out.at[pl.ds(g * CHUNK, CHUNK)])
            return carry

        lax.fori_loop(0, nch, chunk, 0)

    return k


def _make_kD(n_src, n_dst, H, G, halves=1):
    NG = H // G
    W = G * 32
    half = n_dst // halves
    rpt = half // NS
    NBUF = 4 if G > 1 else 8

    @functools.partial(
        pl.kernel, mesh=_mesh, compiler_params=_sc_params,
        out_type=jax.ShapeDtypeStruct((NC * NG * n_dst, W), jnp.float32),
        scratch_types=[
            pltpu.VMEM((NSUB, SUB), jnp.int32),
            pltpu.VMEM((NSUB, SUB), jnp.int32),
            pltpu.VMEM((NSUB, SUB), jnp.int32),
            pltpu.VMEM((NSUB, SUB), jnp.int32),
            pltpu.VMEM((CHUNK, 16), jnp.float32),
            pltpu.VMEM((NBUF * SUB, W), jnp.float32),
            pltpu.VMEM_SHARED((half, W), jnp.float32),
            pltpu.SemaphoreType.DMA,
            pltpu.SemaphoreType.DMA,
        ],
    )
    def k(src_hbm, dst_hbm, alpha_hbm, fsrc_hbm, zeros_hbm, out_hbm,
          src_v, dst_v, idx2_v, dst2_v, a_v, rows_v, acc, s_g, s_s):
        cid = lax.axis_index("c")
        sid = lax.axis_index("s")
        w = cid * NS + sid
        nch = (NCHUNKS - 1 - w) // NW + 1

        def pass_body(t, carry0):
            grp = t // halves
            hh = t % halves
            if True:
                lo = hh * half
                pltpu.sync_copy(zeros_hbm, acc.at[pl.ds(sid * rpt, rpt)])
                plsc.subcore_barrier()

                def chunk(j, carry):
                    g = w + j * NW
                    pltpu.sync_copy(src_hbm.at[pl.ds(g * NSUB, NSUB)], src_v)
                    pltpu.sync_copy(dst_hbm.at[pl.ds(g * NSUB, NSUB)], dst_v)
                    pltpu.sync_copy(alpha_hbm.at[pl.ds(g * CHUNK, CHUNK)], a_v)
                    for sb in range(NSUB):
                        for m in range(NSUB):
                            ii = _iota16() + m * 16
                            v = plsc.load_gather(src_v, [_splat(sb), ii])
                            plsc.store_scatter(idx2_v, [_splat(sb), ii],
                                               v + grp * n_src)
                            if halves > 1:
                                d = plsc.load_gather(dst_v, [_splat(sb), ii]) - lo
                                d = jnp.clip(d, 0, half - 1)
                                plsc.store_scatter(dst2_v, [_splat(sb), ii], d)
                    didx = dst2_v if halves > 1 else dst_v
                    for r in range(NSUB // NBUF):
                        cps = []
                        for b in range(NBUF):
                            sb = r * NBUF + b
                            cps.append(pltpu.async_copy(
                                fsrc_hbm.at[idx2_v.at[sb]],
                                rows_v.at[pl.ds(b * SUB, SUB)], s_g))
                        for cp in cps:
                            cp.wait()
                        scs = []
                        for b in range(NBUF):
                            sb = r * NBUF + b
                            base = g * CHUNK + sb * SUB

                            def edge(i, c2):
                                msk = _edge_mask(base + i)
                                if halves > 1:
                                    dv = plsc.load_gather(
                                        dst_v, [_splat(sb), _splat(i)])
                                    one = jnp.full((16,), 1.0, jnp.float32)
                                    zero = jnp.full((16,), 0.0, jnp.float32)
                                    inr = jnp.where(
                                        (dv >= lo) & (dv < lo + half),
                                        one, zero)
                                    msk = msk * inr
                                row = b * SUB + i
                                for jj in range(G):
                                    av = plsc.load_gather(
                                        a_v, [_splat(sb * SUB + i),
                                              _splat(grp * G + jj)]) * msk
                                    _vwrite(rows_v, row, 32 * jj,
                                            _vread(rows_v, row, 32 * jj) * av)
                                    _vwrite(rows_v, row, 32 * jj + 16,
                                            _vread(rows_v, row, 32 * jj + 16) * av)
                                return c2

                            lax.fori_loop(0, SUB, edge, 0, unroll=2)
                            scs.append(pltpu.async_copy(
                                rows_v.at[pl.ds(b * SUB, SUB)],
                                acc.at[didx.at[sb]], s_s, add=True))
                        for cp in scs:
                            cp.wait()
                    return carry

                lax.fori_loop(0, nch, chunk, 0)
                plsc.subcore_barrier()
                pltpu.sync_copy(
                    acc.at[pl.ds(sid * rpt, rpt)],
                    out_hbm.at[pl.ds((cid * NG + grp) * n_dst + lo + sid * rpt,
                                     rpt)])
                plsc.subcore_barrier()
            return carry0

        lax.fori_loop(0, NG * halves, pass_body, 0)

    return k


def _make_tgt_gather():
    @functools.partial(
        pl.kernel, mesh=_mesh, compiler_params=_sc_params,
        out_type=jax.ShapeDtypeStruct((NC * BS, 32), jnp.float32),
        scratch_types=[
            pltpu.VMEM((SUB,), jnp.int32),
            pltpu.VMEM((SUB,), jnp.int32),
            pltpu.VMEM((SUB, 32), jnp.float32),
            pltpu.SemaphoreType.DMA,
        ],
    )
    def k(parts_hbm, idx_hbm, out_hbm, idx_v, idx2_v, rows_v, sem):
        w = lax.axis_index("c") * NS + lax.axis_index("s")
        pltpu.sync_copy(idx_hbm.at[w], idx_v)
        pltpu.async_copy(parts_hbm.at[idx_v], rows_v, sem).wait()
        pltpu.sync_copy(rows_v, out_hbm.at[pl.ds(w * SUB, SUB)])
        for m in range(NSUB):
            v = plsc.load_gather(idx_v, [_iota16() + m * 16])
            plsc.store_scatter(idx2_v, [_iota16() + m * 16], v + N_ITEM)
        pltpu.async_copy(parts_hbm.at[idx2_v], rows_v, sem).wait()
        pltpu.sync_copy(rows_v, out_hbm.at[pl.ds(BS + w * SUB, SUB)])

    return k


def _tc_item_l1(rows_id, rows_rat, ts, W_item, b_item, W1bt, al1bt, W1hi, ar1hi):
    BN = 2000
    grid = N_ITEM // BN

    def body(id_ref, rat_ref, ts_ref, Wi_ref, bi_ref, Wbt_ref, albt_ref,
             Whi_ref, arhi_ref, f_ref, el_ref, er_ref, mx_ref):
        x = jnp.concatenate([id_ref[...], rat_ref[...], ts_ref[...]], axis=1)
        emb = x @ Wi_ref[...] + bi_ref[...]
        f = emb @ Wbt_ref[...]
        gmat = emb @ Whi_ref[...]
        els = []
        ers = []
        for h in range(H1):
            fh = f[:, 32 * h:32 * h + 32]
            f_ref[h] = fh
            els.append(jnp.sum(fh * albt_ref[h][None, :], axis=1, keepdims=True))
            ers.append(jnp.sum(gmat[:, 32 * h:32 * h + 32] * arhi_ref[h][None, :],
                               axis=1, keepdims=True))
        z = jnp.zeros((BN, 8), jnp.float32)
        el16 = jnp.concatenate(els + [z], axis=1)
        er16 = jnp.concatenate(ers + [z], axis=1)
        el_ref[...] = el16
        er_ref[...] = er16
        m = jnp.concatenate([jnp.max(el16[:, :8], axis=0),
                             jnp.max(er16[:, :8], axis=0)])[None, :]

        @pl.when(pl.program_id(0) == 0)
        def _():
            mx_ref[...] = jnp.full((1, 16), -1e30, jnp.float32)

        mx_ref[...] = jnp.maximum(mx_ref[...], m)

    return pl.pallas_call(
        body,
        grid=(grid,),
        in_specs=[
            pl.BlockSpec((BN, 32), lambda i: (i, 0)),
            pl.BlockSpec((BN, 32), lambda i: (i, 0)),
            pl.BlockSpec((BN, 32), lambda i: (i, 0)),
            pl.BlockSpec((96, 32), lambda i: (0, 0)),
            pl.BlockSpec((1, 32), lambda i: (0, 0)),
            pl.BlockSpec((32, 256), lambda i: (0, 0)),
            pl.BlockSpec((8, 32), lambda i: (0, 0)),
            pl.BlockSpec((32, 256), lambda i: (0, 0)),
            pl.BlockSpec((8, 32), lambda i: (0, 0)),
        ],
        out_specs=[
            pl.BlockSpec((8, BN, 32), lambda i: (0, i, 0)),
            pl.BlockSpec((BN, 16), lambda i: (i, 0)),
            pl.BlockSpec((BN, 16), lambda i: (i, 0)),
            pl.BlockSpec((1, 16), lambda i: (0, 0)),
        ],
        out_shape=[
            jax.ShapeDtypeStruct((8, N_ITEM, 32), jnp.float32),
            jax.ShapeDtypeStruct((N_ITEM, 16), jnp.float32),
            jax.ShapeDtypeStruct((N_ITEM, 16), jnp.float32),
            jax.ShapeDtypeStruct((1, 16), jnp.float32),
        ],
    )(rows_id, rows_rat, ts, W_item, b_item, W1bt, al1bt, W1hi, ar1hi)


def _tc_feat_l1(rows_feat, W_feat, b_feat, W1hi, al1hi, W1bt, ar1bt):
    BN = 2000
    grid = N_FEAT // BN

    def body(rf_ref, Wf_ref, bf_ref, Whi_ref, alhi_ref, Wbt_ref, arbt_ref,
             f_ref, el_ref, er_ref, mx_ref):
        emb = rf_ref[...] @ Wf_ref[...] + bf_ref[...]
        f = emb @ Whi_ref[...]
        gmat = emb @ Wbt_ref[...]
        els = []
        ers = []
        for h in range(H1):
            fh = f[:, 32 * h:32 * h + 32]
            f_ref[h] = fh
            els.append(jnp.sum(fh * alhi_ref[h][None, :], axis=1, keepdims=True))
            ers.append(jnp.sum(gmat[:, 32 * h:32 * h + 32] * arbt_ref[h][None, :],
                               axis=1, keepdims=True))
        z = jnp.zeros((BN, 8), jnp.float32)
        el16 = jnp.concatenate(els + [z], axis=1)
        er16 = jnp.concatenate(ers + [z], axis=1)
        el_ref[...] = el16
        er_ref[...] = er16
        m = jnp.concatenate([jnp.max(el16[:, :8], axis=0),
                             jnp.max(er16[:, :8], axis=0)])[None, :]

        @pl.when(pl.program_id(0) == 0)
        def _():
            mx_ref[...] = jnp.full((1, 16), -1e30, jnp.float32)

        mx_ref[...] = jnp.maximum(mx_ref[...], m)

    return pl.pallas_call(
        body,
        grid=(grid,),
        in_specs=[
            pl.BlockSpec((BN, 32), lambda i: (i, 0)),
            pl.BlockSpec((32, 32), lambda i: (0, 0)),
            pl.BlockSpec((1, 32), lambda i: (0, 0)),
            pl.BlockSpec((32, 256), lambda i: (0, 0)),
            pl.BlockSpec((8, 32), lambda i: (0, 0)),
            pl.BlockSpec((32, 256), lambda i: (0, 0)),
            pl.BlockSpec((8, 32), lambda i: (0, 0)),
        ],
        out_specs=[
            pl.BlockSpec((8, BN, 32), lambda i: (0, i, 0)),
            pl.BlockSpec((BN, 16), lambda i: (i, 0)),
            pl.BlockSpec((BN, 16), lambda i: (i, 0)),
            pl.BlockSpec((1, 16), lambda i: (0, 0)),
        ],
        out_shape=[
            jax.ShapeDtypeStruct((H1, N_FEAT, 32), jnp.float32),
            jax.ShapeDtypeStruct((N_FEAT, 16), jnp.float32),
            jax.ShapeDtypeStruct((N_FEAT, 16), jnp.float32),
            jax.ShapeDtypeStruct((1, 16), jnp.float32),
        ],
    )(rows_feat, W_feat, b_feat, W1hi, al1hi, W1bt, ar1bt)


def _tc_l2_prep(parts, b1, W2a, al2a, W2b, ar2b, n, NG, G):
    BN = 2000
    grid = n // BN
    W = G * 32

    def body(p_ref, b1_ref, Wa_ref, ala_ref, Wb_ref, arb_ref,
             f_ref, el_ref, er_ref, mx_ref):
        cols = []
        for h in range(H1):
            grp, jj = divmod(h, G)
            cols.append(p_ref[0, grp, :, 32 * jj:32 * jj + 32]
                        + p_ref[1, grp, :, 32 * jj:32 * jj + 32])
        hmat = jnp.concatenate(cols, axis=1) + b1_ref[...]
        f2 = hmat @ Wa_ref[...]
        g2 = hmat @ Wb_ref[...]
        f_ref[...] = f2
        el = jnp.sum(f2 * ala_ref[0][None, :], axis=1, keepdims=True)
        er = jnp.sum(g2 * arb_ref[0][None, :], axis=1, keepdims=True)
        z = jnp.zeros((BN, 15), jnp.float32)
        el16 = jnp.concatenate([el, z], axis=1)
        er16 = jnp.concatenate([er, z], axis=1)
        el_ref[...] = el16
        er_ref[...] = er16
        m = jnp.concatenate(
            [jnp.max(el16[:, :8], axis=0), jnp.max(er16[:, :8], axis=0)])[None, :]

        @pl.when(pl.program_id(0) == 0)
        def _():
            mx_ref[...] = jnp.full((1, 16), -1e30, jnp.float32)

        mx_ref[...] = jnp.maximum(mx_ref[...], m)

    return pl.pallas_call(
        body,
        grid=(grid,),
        in_specs=[
            pl.BlockSpec((2, NG, BN, W), lambda i: (0, 0, i, 0)),
            pl.BlockSpec((1, 256), lambda i: (0, 0)),
            pl.BlockSpec((256, 32), lambda i: (0, 0)),
            pl.BlockSpec((1, 32), lambda i: (0, 0)),
            pl.BlockSpec((256, 32), lambda i: (0, 0)),
            pl.BlockSpec((1, 32), lambda i: (0, 0)),
        ],
        out_specs=[
            pl.BlockSpec((BN, 32), lambda i: (i, 0)),
            pl.BlockSpec((BN, 16), lambda i: (i, 0)),
            pl.BlockSpec((BN, 16), lambda i: (i, 0)),
            pl.BlockSpec((1, 16), lambda i: (0, 0)),
        ],
        out_shape=[
            jax.ShapeDtypeStruct((n, 32), jnp.float32),
            jax.ShapeDtypeStruct((n, 16), jnp.float32),
            jax.ShapeDtypeStruct((n, 16), jnp.float32),
            jax.ShapeDtypeStruct((1, 16), jnp.float32),
        ],
    )(parts, b1, W2a, al2a, W2b, ar2b)


def _tc_final(rows_user, W_user, b_user, tgt_parts, b2hi):
    def body(ru_ref, Wu_ref, bu_ref, tp_ref, b2_ref, o_ref):
        u = ru_ref[...] @ Wu_ref[...] + bu_ref[...]
        t = tp_ref[0] + tp_ref[1] + b2_ref[...]
        num = jnp.sum(u * t, axis=1)
        nu = jnp.maximum(jnp.sqrt(jnp.sum(u * u, axis=1)), 1e-8)
        nt = jnp.maximum(jnp.sqrt(jnp.sum(t * t, axis=1)), 1e-8)
        o_ref[...] = jax.nn.sigmoid(num / (nu * nt))

    return pl.pallas_call(
        body,
        out_shape=jax.ShapeDtypeStruct((BS,), jnp.float32),
    )(rows_user, W_user, b_user, tgt_parts, b2hi)


def _pad_edges(x):
    return jnp.pad(x, (0, E_PAD - E)).reshape(E_PAD // SUB, SUB).astype(jnp.int32)


def _c16(m, nh):
    c = jax.nn.leaky_relu(m, 0.2)
    return jnp.concatenate([c[:nh], jnp.full((16 - nh,), 100.0, jnp.float32)])


def kernel(user_feat, item_id, item_rating, feat_id, is_target,
           edge_bt_src, edge_bt_dst, edge_hi_src, edge_hi_dst,
           ts_emb, emb_table, W_user, b_user, W_item, b_item, W_feat, b_feat,
           W1_bt, al1_bt, ar1_bt, b1_bt, W1_hi, al1_hi, ar1_hi, b1_hi,
           W2_bt, al2_bt, ar2_bt, b2_bt, W2_hi, al2_hi, ar2_hi, b2_hi):
    zeros16 = jnp.zeros((2500, 16), jnp.float32)

    n_all = N_ITEM + N_FEAT + BS + N_ITEM
    n_all_pad = ((n_all + NW * SUB - 1) // (NW * SUB)) * (NW * SUB)
    idx_all = jnp.concatenate([
        item_id.astype(jnp.int32), feat_id.astype(jnp.int32),
        user_feat[:, 0].astype(jnp.int32), item_rating.astype(jnp.int32),
        jnp.zeros((n_all_pad - n_all,), jnp.int32)]).reshape(n_all_pad // SUB, SUB)
    rows = _make_gather(n_all_pad, emb_table.shape[0])(emb_table, idx_all)
    rows_id = rows[:N_ITEM]
    rows_feat = rows[N_ITEM:N_ITEM + N_FEAT]
    rows_user = rows[N_ITEM + N_FEAT:N_ITEM + N_FEAT + BS]
    rows_rat = rows[N_ITEM + N_FEAT + BS:n_all]

    f1_bt, el1bt, er1hi, mx_i = _tc_item_l1(
        rows_id, rows_rat, ts_emb, W_item, b_item.reshape(1, 32),
        W1_bt, al1_bt, W1_hi, ar1_hi)
    f1_hi, el1hi, er1bt, mx_f = _tc_feat_l1(
        rows_feat, W_feat, b_feat.reshape(1, 32), W1_hi, al1_hi, W1_bt, ar1_bt)

    c_bt1 = _c16(mx_i[0, :8] + mx_f[0, 8:], 8)
    c_hi1 = _c16(mx_f[0, :8] + mx_i[0, 8:], 8)

    sbt = _pad_edges(edge_bt_src)
    dbt = _pad_edges(edge_bt_dst)
    shi = _pad_edges(edge_hi_src)
    dhi = _pad_edges(edge_hi_dst)

    den_bt = _make_kA(N_FEAT)(sbt, dbt, el1bt, er1bt, c_bt1, zeros16)
    den_hi = _make_kA(N_ITEM)(shi, dhi, el1hi, er1hi, c_hi1, zeros16)
    alpha_bt = _make_kB()(sbt, dbt, el1bt, er1bt,
                          den_bt[:N_FEAT], den_bt[N_FEAT:], c_bt1)
    alpha_hi = _make_kB()(shi, dhi, el1hi, er1hi,
                          den_hi[:N_ITEM], den_hi[N_ITEM:], c_hi1)
    bt_parts = _make_kD(N_ITEM, N_FEAT, H1, 1)(
        sbt, dbt, alpha_bt, f1_bt.reshape(H1 * N_ITEM, 32),
        jnp.zeros((N_FEAT // NS, 32), jnp.float32))
    hi_parts = _make_kD(N_FEAT, N_ITEM, H1, 1, halves=2)(
        shi, dhi, alpha_hi, f1_hi.reshape(H1 * N_FEAT, 32),
        jnp.zeros((N_ITEM // (2 * NS), 32), jnp.float32))

    f2_bt, el2bt, er2hi, mx2_i = _tc_l2_prep(
        hi_parts.reshape(2, H1, N_ITEM, 32), b1_hi.reshape(1, 256),
        W2_bt, al2_bt, W2_hi, ar2_hi, N_ITEM, H1, 1)
    f2_hi, el2hi, er2bt, mx2_f = _tc_l2_prep(
        bt_parts.reshape(2, H1, N_FEAT, 32), b1_bt.reshape(1, 256),
        W2_hi, al2_hi, W2_bt, ar2_bt, N_FEAT, H1, 1)

    c_hi2 = _c16(mx2_f[0, :8] + mx2_i[0, 8:], 1)

    den2_hi = _make_kA(N_ITEM)(shi, dhi, el2hi, er2hi, c_hi2, zeros16)
    alpha2_hi = _make_kB()(shi, dhi, el2hi, er2hi,
                           den2_hi[:N_ITEM], den2_hi[N_ITEM:], c_hi2)
    hi2_parts = _make_kD(N_FEAT, N_ITEM, 1, 1, halves=2)(
        shi, dhi, alpha2_hi, f2_hi,
        jnp.zeros((N_ITEM // (2 * NS), 32), jnp.float32))

    idx_tgt = jnp.nonzero(is_target, size=BS, fill_value=0)[0].astype(jnp.int32)
    tgt_parts = _make_tgt_gather()(hi2_parts, idx_tgt.reshape(NW, SUB))
    return _tc_final(rows_user, W_user, b_user.reshape(1, 32),
                     tgt_parts.reshape(2, BS, 32), b2_hi.reshape(1, 32))


# speedup vs baseline: 1.0168x; 1.0168x over previous
import functools

import jax
import jax.numpy as jnp
from jax import lax
from jax.experimental import pallas as pl
from jax.experimental.pallas import tpu as pltpu
from jax.experimental.pallas import tpu_sc as plsc

N_ITEM = 40000
N_FEAT = 10000
E = 400000
BS = 4096
EMB = 32
H1 = 8
OUT = 32

NC = 2
NS = 16
NW = NC * NS
SUB = 128
NSUB = 8
CHUNK = SUB * NSUB
E_PAD = ((E + CHUNK - 1) // CHUNK) * CHUNK
NCHUNKS = E_PAD // CHUNK

_mesh = plsc.VectorSubcoreMesh(core_axis_name="c", subcore_axis_name="s")
_sc_params = pltpu.CompilerParams(use_tc_tiling_on_sc=False,
                                  needs_layout_passes=False)


def _iota16():
    return lax.iota(jnp.int32, 16)


def _splat(x):
    return jnp.full((16,), x, jnp.int32)


def _vread(ref, i, off):
    return plsc.load_gather(ref, [_splat(i), _iota16() + off])


def _vwrite(ref, i, off, val):
    plsc.store_scatter(ref, [_splat(i), _iota16() + off], val)


def _edge_mask(gid):
    one = jnp.full((16,), 1.0, jnp.float32)
    zero = jnp.full((16,), 0.0, jnp.float32)
    return jnp.where(_splat(gid) < E, one, zero)


def _make_gather(n_rows_pad, table_rows):
    nloc = n_rows_pad // (NW * SUB)

    @functools.partial(
        pl.kernel, mesh=_mesh, compiler_params=_sc_params,
        out_type=jax.ShapeDtypeStruct((n_rows_pad, 32), jnp.float32),
        scratch_types=[
            pltpu.VMEM((SUB,), jnp.int32),
            pltpu.VMEM((SUB, 32), jnp.float32),
            pltpu.SemaphoreType.DMA,
        ],
    )
    def k(table_hbm, idx_hbm, out_hbm, idx_v, rows_v, sem):
        w = lax.axis_index("c") * NS + lax.axis_index("s")

        def body(j, carry):
            r = w * nloc + j
            pltpu.sync_copy(idx_hbm.at[r], idx_v)
            pltpu.async_copy(table_hbm.at[idx_v], rows_v, sem).wait()
            pltpu.sync_copy(rows_v, out_hbm.at[pl.ds(r * SUB, SUB)])
            return carry

        lax.fori_loop(0, nloc, body, 0)

    return k


def _make_kA(n_dst):
    rpt = n_dst // NS

    @functools.partial(
        pl.kernel, mesh=_mesh, compiler_params=_sc_params,
        out_type=jax.ShapeDtypeStruct((NC * n_dst, 16), jnp.float32),
        scratch_types=[
            pltpu.VMEM((NSUB, SUB), jnp.int32),
            pltpu.VMEM((NSUB, SUB), jnp.int32),
            pltpu.VMEM((CHUNK, 16), jnp.float32),
            pltpu.VMEM((CHUNK, 16), jnp.float32),
            pltpu.VMEM((CHUNK, 16), jnp.float32),
            pltpu.VMEM((16,), jnp.float32),
            pltpu.VMEM_SHARED((n_dst, 16), jnp.float32),
            pltpu.SemaphoreType.DMA,
            pltpu.SemaphoreType.DMA,
            pltpu.SemaphoreType.DMA,
        ],
    )
    def k(src_hbm, dst_hbm, el_hbm, er_hbm, c_hbm, zeros_hbm, den_out,
          src_v, dst_v, el_v, er_v, ee_v, c_v, acc, s_el, s_er, s_sc):
        cid = lax.axis_index("c")
        sid = lax.axis_index("s")
        w = cid * NS + sid
        pltpu.sync_copy(zeros_hbm.at[pl.ds(0, rpt)], acc.at[pl.ds(sid * rpt, rpt)])
        pltpu.sync_copy(c_hbm, c_v)
        plsc.subcore_barrier()
        nch = (NCHUNKS - 1 - w) // NW + 1

        def chunk(j, carry):
            g = w + j * NW
            pltpu.sync_copy(src_hbm.at[pl.ds(g * NSUB, NSUB)], src_v)
            pltpu.sync_copy(dst_hbm.at[pl.ds(g * NSUB, NSUB)], dst_v)
            cps = []
            for sb in range(NSUB):
                cps.append(pltpu.async_copy(
                    el_hbm.at[src_v.at[sb]],
                    el_v.at[pl.ds(sb * SUB, SUB)], s_el))
                cps.append(pltpu.async_copy(
                    er_hbm.at[dst_v.at[sb]],
                    er_v.at[pl.ds(sb * SUB, SUB)], s_er))
            for cp in cps:
                cp.wait()
            cvec = c_v[...]

            def edge(i, c2):
                ev = _vread(el_v, i, 0) + _vread(er_v, i, 0)
                ev = jnp.maximum(ev, 0.2 * ev)
                ee = jnp.exp(ev - cvec) * _edge_mask(g * CHUNK + i)
                _vwrite(ee_v, i, 0, ee)
                return c2

            lax.fori_loop(0, CHUNK, edge, 0, unroll=2)
            scs = []
            for sb in range(NSUB):
                scs.append(pltpu.async_copy(
                    ee_v.at[pl.ds(sb * SUB, SUB)],
                    acc.at[dst_v.at[sb]], s_sc, add=True))
            for cp in scs:
                cp.wait()
            return carry

        lax.fori_loop(0, nch, chunk, 0)
        plsc.subcore_barrier()
        pltpu.sync_copy(acc.at[pl.ds(sid * rpt, rpt)],
                        den_out.at[pl.ds(cid * n_dst + sid * rpt, rpt)])

    return k


def _make_kB():
    @functools.partial(
        pl.kernel, mesh=_mesh, compiler_params=_sc_params,
        out_type=jax.ShapeDtypeStruct((E_PAD, 16), jnp.float32),
        scratch_types=[
            pltpu.VMEM((NSUB, SUB), jnp.int32),
            pltpu.VMEM((NSUB, SUB), jnp.int32),
            pltpu.VMEM((CHUNK, 16), jnp.float32),
            pltpu.VMEM((CHUNK, 16), jnp.float32),
            pltpu.VMEM((CHUNK, 16), jnp.float32),
            pltpu.VMEM((CHUNK, 16), jnp.float32),
            pltpu.VMEM((CHUNK, 16), jnp.float32),
            pltpu.VMEM((16,), jnp.float32),
            pltpu.SemaphoreType.DMA,
            pltpu.SemaphoreType.DMA,
            pltpu.SemaphoreType.DMA,
            pltpu.SemaphoreType.DMA,
        ],
    )
    def k(src_hbm, dst_hbm, el_hbm, er_hbm, d0_hbm, d1_hbm, c_hbm, alpha_out,
          src_v, dst_v, el_v, er_v, d0_v, d1_v, a_v, c_v, s1, s2, s3, s4):
        cid = lax.axis_index("c")
        sid = lax.axis_index("s")
        w = cid * NS + sid
        pltpu.sync_copy(c_hbm, c_v)
        nch = (NCHUNKS - 1 - w) // NW + 1
        eps = jnp.full((16,), 1e-16, jnp.float32)

        def chunk(j, carry):
            g = w + j * NW
            pltpu.sync_copy(src_hbm.at[pl.ds(g * NSUB, NSUB)], src_v)
            pltpu.sync_copy(dst_hbm.at[pl.ds(g * NSUB, NSUB)], dst_v)
            cps = []
            for sb in range(NSUB):
                cps.append(pltpu.async_copy(
                    el_hbm.at[src_v.at[sb]],
                    el_v.at[pl.ds(sb * SUB, SUB)], s1))
                cps.append(pltpu.async_copy(
                    er_hbm.at[dst_v.at[sb]],
                    er_v.at[pl.ds(sb * SUB, SUB)], s2))
                cps.append(pltpu.async_copy(
                    d0_hbm.at[dst_v.at[sb]],
                    d0_v.at[pl.ds(sb * SUB, SUB)], s3))
                cps.append(pltpu.async_copy(
                    d1_hbm.at[dst_v.at[sb]],
                    d1_v.at[pl.ds(sb * SUB, SUB)], s4))
            for cp in cps:
                cp.wait()
            cvec = c_v[...]

            def edge(i, c2):
                ev = _vread(el_v, i, 0) + _vread(er_v, i, 0)
                ev = jnp.maximum(ev, 0.2 * ev)
                ee = jnp.exp(ev - cvec)
                den = _vread(d0_v, i, 0) + _vread(d1_v, i, 0)
                alpha = ee / jnp.maximum(den, eps)
                _vwrite(a_v, i, 0, alpha)
                return c2

            lax.fori_loop(0, CHUNK, edge, 0, unroll=2)
            pltpu.sync_copy(a_v, alpha_out.at[pl.ds(g * CHUNK, CHUNK)])
            return carry

        lax.fori_loop(0, nch, chunk, 0)

    return k


def _make_kD(n_src, n_dst, H, G, halves=1):
    NG = H // G
    W = G * 32
    half = n_dst // halves
    rpt = half // NS
    NBUF = 4 if G > 1 else 8

    @functools.partial(
        pl.kernel, mesh=_mesh, compiler_params=_sc_params,
        out_type=jax.ShapeDtypeStruct((NC * NG * n_dst, W), jnp.float32),
        scratch_types=[
            pltpu.VMEM((NSUB, SUB), jnp.int32),
            pltpu.VMEM((NSUB, SUB), jnp.int32),
            pltpu.VMEM((NSUB, SUB), jnp.int32),
            pltpu.VMEM((NSUB, SUB), jnp.int32),
            pltpu.VMEM((CHUNK, 16), jnp.float32),
            pltpu.VMEM((NBUF * SUB, W), jnp.float32),
            pltpu.VMEM_SHARED((half, W), jnp.float32),
            pltpu.SemaphoreType.DMA,
            pltpu.SemaphoreType.DMA,
        ],
    )
    def k(src_hbm, dst_hbm, alpha_hbm, fsrc_hbm, zeros_hbm, out_hbm,
          src_v, dst_v, idx2_v, dst2_v, a_v, rows_v, acc, s_g, s_s):
        cid = lax.axis_index("c")
        sid = lax.axis_index("s")
        w = cid * NS + sid
        nch = (NCHUNKS - 1 - w) // NW + 1

        def pass_body(t, carry0):
            grp = t // halves
            hh = t % halves
            if True:
                lo = hh * half
                pltpu.sync_copy(zeros_hbm, acc.at[pl.ds(sid * rpt, rpt)])
                plsc.subcore_barrier()

                def chunk(j, carry):
                    g = w + j * NW
                    pltpu.sync_copy(src_hbm.at[pl.ds(g * NSUB, NSUB)], src_v)
                    pltpu.sync_copy(dst_hbm.at[pl.ds(g * NSUB, NSUB)], dst_v)
                    pltpu.sync_copy(alpha_hbm.at[pl.ds(g * CHUNK, CHUNK)], a_v)
                    for sb in range(NSUB):
                        for m in range(NSUB):
                            ii = _iota16() + m * 16
                            v = plsc.load_gather(src_v, [_splat(sb), ii])
                            plsc.store_scatter(idx2_v, [_splat(sb), ii],
                                               v + grp * n_src)
                            if halves > 1:
                                d = plsc.load_gather(dst_v, [_splat(sb), ii]) - lo
                                d = jnp.clip(d, 0, half - 1)
                                plsc.store_scatter(dst2_v, [_splat(sb), ii], d)
                    didx = dst2_v if halves > 1 else dst_v
                    for r in range(NSUB // NBUF):
                        cps = []
                        for b in range(NBUF):
                            sb = r * NBUF + b
                            cps.append(pltpu.async_copy(
                                fsrc_hbm.at[idx2_v.at[sb]],
                                rows_v.at[pl.ds(b * SUB, SUB)], s_g))
                        for cp in cps:
                            cp.wait()
                        scs = []
                        for b in range(NBUF):
                            sb = r * NBUF + b
                            base = g * CHUNK + sb * SUB

                            def edge(i, c2):
                                msk = _edge_mask(base + i)
                                if halves > 1:
                                    dv = plsc.load_gather(
                                        dst_v, [_splat(sb), _splat(i)])
                                    one = jnp.full((16,), 1.0, jnp.float32)
                                    zero = jnp.full((16,), 0.0, jnp.float32)
                                    inr = jnp.where(
                                        (dv >= lo) & (dv < lo + half),
                                        one, zero)
                                    msk = msk * inr
                                row = b * SUB + i
                                for jj in range(G):
                                    av = plsc.load_gather(
                                        a_v, [_splat(sb * SUB + i),
                                              _splat(grp * G + jj)]) * msk
                                    _vwrite(rows_v, row, 32 * jj,
                                            _vread(rows_v, row, 32 * jj) * av)
                                    _vwrite(rows_v, row, 32 * jj + 16,
                                            _vread(rows_v, row, 32 * jj + 16) * av)
                                return c2

                            lax.fori_loop(0, SUB, edge, 0, unroll=2)
                            scs.append(pltpu.async_copy(
                                rows_v.at[pl.ds(b * SUB, SUB)],
                                acc.at[didx.at[sb]], s_s, add=True))
                        for cp in scs:
                            cp.wait()
                    return carry

                lax.fori_loop(0, nch, chunk, 0)
                plsc.subcore_barrier()
                pltpu.sync_copy(
                    acc.at[pl.ds(sid * rpt, rpt)],
                    out_hbm.at[pl.ds((cid * NG + grp) * n_dst + lo + sid * rpt,
                                     rpt)])
                plsc.subcore_barrier()
            return carry0

        lax.fori_loop(0, NG * halves, pass_body, 0)

    return k


def _make_tgt_gather():
    @functools.partial(
        pl.kernel, mesh=_mesh, compiler_params=_sc_params,
        out_type=jax.ShapeDtypeStruct((NC * BS, 32), jnp.float32),
        scratch_types=[
            pltpu.VMEM((SUB,), jnp.int32),
            pltpu.VMEM((SUB,), jnp.int32),
            pltpu.VMEM((SUB, 32), jnp.float32),
            pltpu.SemaphoreType.DMA,
        ],
    )
    def k(parts_hbm, idx_hbm, out_hbm, idx_v, idx2_v, rows_v, sem):
        w = lax.axis_index("c") * NS + lax.axis_index("s")
        pltpu.sync_copy(idx_hbm.at[w], idx_v)
        pltpu.async_copy(parts_hbm.at[idx_v], rows_v, sem).wait()
        pltpu.sync_copy(rows_v, out_hbm.at[pl.ds(w * SUB, SUB)])
        for m in range(NSUB):
            v = plsc.load_gather(idx_v, [_iota16() + m * 16])
            plsc.store_scatter(idx2_v, [_iota16() + m * 16], v + N_ITEM)
        pltpu.async_copy(parts_hbm.at[idx2_v], rows_v, sem).wait()
        pltpu.sync_copy(rows_v, out_hbm.at[pl.ds(BS + w * SUB, SUB)])

    return k


def _tc_item_l1(rows_id, rows_rat, ts, W_item, b_item, W1bt, al1bt, W1hi, ar1hi):
    BN = 2000
    grid = N_ITEM // BN

    def body(id_ref, rat_ref, ts_ref, Wi_ref, bi_ref, Wbt_ref, albt_ref,
             Whi_ref, arhi_ref, f_ref, el_ref, er_ref, mx_ref):
        x = jnp.concatenate([id_ref[...], rat_ref[...], ts_ref[...]], axis=1)
        emb = x @ Wi_ref[...] + bi_ref[...]
        f = emb @ Wbt_ref[...]
        gmat = emb @ Whi_ref[...]
        els = []
        ers = []
        for h in range(H1):
            fh = f[:, 32 * h:32 * h + 32]
            grp, jj = divmod(h, 2)
            f_ref[grp, :, 32 * jj:32 * jj + 32] = fh
            els.append(jnp.sum(fh * albt_ref[h][None, :], axis=1, keepdims=True))
            ers.append(jnp.sum(gmat[:, 32 * h:32 * h + 32] * arhi_ref[h][None, :],
                               axis=1, keepdims=True))
        z = jnp.zeros((BN, 8), jnp.float32)
        el16 = jnp.concatenate(els + [z], axis=1)
        er16 = jnp.concatenate(ers + [z], axis=1)
        el_ref[...] = el16
        er_ref[...] = er16
        m = jnp.concatenate([jnp.max(el16[:, :8], axis=0),
                             jnp.max(er16[:, :8], axis=0)])[None, :]

        @pl.when(pl.program_id(0) == 0)
        def _():
            mx_ref[...] = jnp.full((1, 16), -1e30, jnp.float32)

        mx_ref[...] = jnp.maximum(mx_ref[...], m)

    return pl.pallas_call(
        body,
        grid=(grid,),
        in_specs=[
            pl.BlockSpec((BN, 32), lambda i: (i, 0)),
            pl.BlockSpec((BN, 32), lambda i: (i, 0)),
            pl.BlockSpec((BN, 32), lambda i: (i, 0)),
            pl.BlockSpec((96, 32), lambda i: (0, 0)),
            pl.BlockSpec((1, 32), lambda i: (0, 0)),
            pl.BlockSpec((32, 256), lambda i: (0, 0)),
            pl.BlockSpec((8, 32), lambda i: (0, 0)),
            pl.BlockSpec((32, 256), lambda i: (0, 0)),
            pl.BlockSpec((8, 32), lambda i: (0, 0)),
        ],
        out_specs=[
            pl.BlockSpec((4, BN, 64), lambda i: (0, i, 0)),
            pl.BlockSpec((BN, 16), lambda i: (i, 0)),
            pl.BlockSpec((BN, 16), lambda i: (i, 0)),
            pl.BlockSpec((1, 16), lambda i: (0, 0)),
        ],
        out_shape=[
            jax.ShapeDtypeStruct((4, N_ITEM, 64), jnp.float32),
            jax.ShapeDtypeStruct((N_ITEM, 16), jnp.float32),
            jax.ShapeDtypeStruct((N_ITEM, 16), jnp.float32),
            jax.ShapeDtypeStruct((1, 16), jnp.float32),
        ],
    )(rows_id, rows_rat, ts, W_item, b_item, W1bt, al1bt, W1hi, ar1hi)


def _tc_feat_l1(rows_feat, W_feat, b_feat, W1hi, al1hi, W1bt, ar1bt):
    BN = 2000
    grid = N_FEAT // BN

    def body(rf_ref, Wf_ref, bf_ref, Whi_ref, alhi_ref, Wbt_ref, arbt_ref,
             f_ref, el_ref, er_ref, mx_ref):
        emb = rf_ref[...] @ Wf_ref[...] + bf_ref[...]
        f = emb @ Whi_ref[...]
        gmat = emb @ Wbt_ref[...]
        els = []
        ers = []
        for h in range(H1):
            fh = f[:, 32 * h:32 * h + 32]
            f_ref[h] = fh
            els.append(jnp.sum(fh * alhi_ref[h][None, :], axis=1, keepdims=True))
            ers.append(jnp.sum(gmat[:, 32 * h:32 * h + 32] * arbt_ref[h][None, :],
                               axis=1, keepdims=True))
        z = jnp.zeros((BN, 8), jnp.float32)
        el16 = jnp.concatenate(els + [z], axis=1)
        er16 = jnp.concatenate(ers + [z], axis=1)
        el_ref[...] = el16
        er_ref[...] = er16
        m = jnp.concatenate([jnp.max(el16[:, :8], axis=0),
                             jnp.max(er16[:, :8], axis=0)])[None, :]

        @pl.when(pl.program_id(0) == 0)
        def _():
            mx_ref[...] = jnp.full((1, 16), -1e30, jnp.float32)

        mx_ref[...] = jnp.maximum(mx_ref[...], m)

    return pl.pallas_call(
        body,
        grid=(grid,),
        in_specs=[
            pl.BlockSpec((BN, 32), lambda i: (i, 0)),
            pl.BlockSpec((32, 32), lambda i: (0, 0)),
            pl.BlockSpec((1, 32), lambda i: (0, 0)),
            pl.BlockSpec((32, 256), lambda i: (0, 0)),
            pl.BlockSpec((8, 32), lambda i: (0, 0)),
            pl.BlockSpec((32, 256), lambda i: (0, 0)),
            pl.BlockSpec((8, 32), lambda i: (0, 0)),
        ],
        out_specs=[
            pl.BlockSpec((8, BN, 32), lambda i: (0, i, 0)),
            pl.BlockSpec((BN, 16), lambda i: (i, 0)),
            pl.BlockSpec((BN, 16), lambda i: (i, 0)),
            pl.BlockSpec((1, 16), lambda i: (0, 0)),
        ],
        out_shape=[
            jax.ShapeDtypeStruct((H1, N_FEAT, 32), jnp.float32),
            jax.ShapeDtypeStruct((N_FEAT, 16), jnp.float32),
            jax.ShapeDtypeStruct((N_FEAT, 16), jnp.float32),
            jax.ShapeDtypeStruct((1, 16), jnp.float32),
        ],
    )(rows_feat, W_feat, b_feat, W1hi, al1hi, W1bt, ar1bt)


def _tc_l2_prep(parts, b1, W2a, al2a, W2b, ar2b, n, NG, G):
    BN = 2000
    grid = n // BN
    W = G * 32

    def body(p_ref, b1_ref, Wa_ref, ala_ref, Wb_ref, arb_ref,
             f_ref, el_ref, er_ref, mx_ref):
        cols = []
        for h in range(H1):
            grp, jj = divmod(h, G)
            cols.append(p_ref[0, grp, :, 32 * jj:32 * jj + 32]
                        + p_ref[1, grp, :, 32 * jj:32 * jj + 32])
        hmat = jnp.concatenate(cols, axis=1) + b1_ref[...]
        f2 = hmat @ Wa_ref[...]
        g2 = hmat @ Wb_ref[...]
        f_ref[...] = f2
        el = jnp.sum(f2 * ala_ref[0][None, :], axis=1, keepdims=True)
        er = jnp.sum(g2 * arb_ref[0][None, :], axis=1, keepdims=True)
        z = jnp.zeros((BN, 15), jnp.float32)
        el16 = jnp.concatenate([el, z], axis=1)
        er16 = jnp.concatenate([er, z], axis=1)
        el_ref[...] = el16
        er_ref[...] = er16
        m = jnp.concatenate(
            [jnp.max(el16[:, :8], axis=0), jnp.max(er16[:, :8], axis=0)])[None, :]

        @pl.when(pl.program_id(0) == 0)
        def _():
            mx_ref[...] = jnp.full((1, 16), -1e30, jnp.float32)

        mx_ref[...] = jnp.maximum(mx_ref[...], m)

    return pl.pallas_call(
        body,
        grid=(grid,),
        in_specs=[
            pl.BlockSpec((2, NG, BN, W), lambda i: (0, 0, i, 0)),
            pl.BlockSpec((1, 256), lambda i: (0, 0)),
            pl.BlockSpec((256, 32), lambda i: (0, 0)),
            pl.BlockSpec((1, 32), lambda i: (0, 0)),
            pl.BlockSpec((256, 32), lambda i: (0, 0)),
            pl.BlockSpec((1, 32), lambda i: (0, 0)),
        ],
        out_specs=[
            pl.BlockSpec((BN, 32), lambda i: (i, 0)),
            pl.BlockSpec((BN, 16), lambda i: (i, 0)),
            pl.BlockSpec((BN, 16), lambda i: (i, 0)),
            pl.BlockSpec((1, 16), lambda i: (0, 0)),
        ],
        out_shape=[
            jax.ShapeDtypeStruct((n, 32), jnp.float32),
            jax.ShapeDtypeStruct((n, 16), jnp.float32),
            jax.ShapeDtypeStruct((n, 16), jnp.float32),
            jax.ShapeDtypeStruct((1, 16), jnp.float32),
        ],
    )(parts, b1, W2a, al2a, W2b, ar2b)


def _tc_final(rows_user, W_user, b_user, tgt_parts, b2hi):
    def body(ru_ref, Wu_ref, bu_ref, tp_ref, b2_ref, o_ref):
        u = ru_ref[...] @ Wu_ref[...] + bu_ref[...]
        t = tp_ref[0] + tp_ref[1] + b2_ref[...]
        num = jnp.sum(u * t, axis=1)
        nu = jnp.maximum(jnp.sqrt(jnp.sum(u * u, axis=1)), 1e-8)
        nt = jnp.maximum(jnp.sqrt(jnp.sum(t * t, axis=1)), 1e-8)
        o_ref[...] = jax.nn.sigmoid(num / (nu * nt))

    return pl.pallas_call(
        body,
        out_shape=jax.ShapeDtypeStruct((BS,), jnp.float32),
    )(rows_user, W_user, b_user, tgt_parts, b2hi)


def _pad_edges(x):
    return jnp.pad(x, (0, E_PAD - E)).reshape(E_PAD // SUB, SUB).astype(jnp.int32)


def _c16(m, nh):
    c = jax.nn.leaky_relu(m, 0.2)
    return jnp.concatenate([c[:nh], jnp.full((16 - nh,), 100.0, jnp.float32)])


def kernel(user_feat, item_id, item_rating, feat_id, is_target,
           edge_bt_src, edge_bt_dst, edge_hi_src, edge_hi_dst,
           ts_emb, emb_table, W_user, b_user, W_item, b_item, W_feat, b_feat,
           W1_bt, al1_bt, ar1_bt, b1_bt, W1_hi, al1_hi, ar1_hi, b1_hi,
           W2_bt, al2_bt, ar2_bt, b2_bt, W2_hi, al2_hi, ar2_hi, b2_hi):
    zeros16 = jnp.zeros((2500, 16), jnp.float32)

    n_all = N_ITEM + N_FEAT + BS + N_ITEM
    n_all_pad = ((n_all + NW * SUB - 1) // (NW * SUB)) * (NW * SUB)
    idx_all = jnp.concatenate([
        item_id.astype(jnp.int32), feat_id.astype(jnp.int32),
        user_feat[:, 0].astype(jnp.int32), item_rating.astype(jnp.int32),
        jnp.zeros((n_all_pad - n_all,), jnp.int32)]).reshape(n_all_pad // SUB, SUB)
    rows = _make_gather(n_all_pad, emb_table.shape[0])(emb_table, idx_all)
    rows_id = rows[:N_ITEM]
    rows_feat = rows[N_ITEM:N_ITEM + N_FEAT]
    rows_user = rows[N_ITEM + N_FEAT:N_ITEM + N_FEAT + BS]
    rows_rat = rows[N_ITEM + N_FEAT + BS:n_all]

    f1_bt, el1bt, er1hi, mx_i = _tc_item_l1(
        rows_id, rows_rat, ts_emb, W_item, b_item.reshape(1, 32),
        W1_bt, al1_bt, W1_hi, ar1_hi)
    f1_hi, el1hi, er1bt, mx_f = _tc_feat_l1(
        rows_feat, W_feat, b_feat.reshape(1, 32), W1_hi, al1_hi, W1_bt, ar1_bt)

    c_bt1 = _c16(mx_i[0, :8] + mx_f[0, 8:], 8)
    c_hi1 = _c16(mx_f[0, :8] + mx_i[0, 8:], 8)

    sbt = _pad_edges(edge_bt_src)
    dbt = _pad_edges(edge_bt_dst)
    shi = _pad_edges(edge_hi_src)
    dhi = _pad_edges(edge_hi_dst)

    den_bt = _make_kA(N_FEAT)(sbt, dbt, el1bt, er1bt, c_bt1, zeros16)
    den_hi = _make_kA(N_ITEM)(shi, dhi, el1hi, er1hi, c_hi1, zeros16)
    alpha_bt = _make_kB()(sbt, dbt, el1bt, er1bt,
                          den_bt[:N_FEAT], den_bt[N_FEAT:], c_bt1)
    alpha_hi = _make_kB()(shi, dhi, el1hi, er1hi,
                          den_hi[:N_ITEM], den_hi[N_ITEM:], c_hi1)
    bt_parts = _make_kD(N_ITEM, N_FEAT, H1, 2)(
        sbt, dbt, alpha_bt, f1_bt.reshape(4 * N_ITEM, 64),
        jnp.zeros((N_FEAT // NS, 64), jnp.float32))
    hi_parts = _make_kD(N_FEAT, N_ITEM, H1, 1, halves=2)(
        shi, dhi, alpha_hi, f1_hi.reshape(H1 * N_FEAT, 32),
        jnp.zeros((N_ITEM // (2 * NS), 32), jnp.float32))

    f2_bt, el2bt, er2hi, mx2_i = _tc_l2_prep(
        hi_parts.reshape(2, H1, N_ITEM, 32), b1_hi.reshape(1, 256),
        W2_bt, al2_bt, W2_hi, ar2_hi, N_ITEM, H1, 1)
    f2_hi, el2hi, er2bt, mx2_f = _tc_l2_prep(
        bt_parts.reshape(2, 4, N_FEAT, 64), b1_bt.reshape(1, 256),
        W2_hi, al2_hi, W2_bt, ar2_bt, N_FEAT, 4, 2)

    c_hi2 = _c16(mx2_f[0, :8] + mx2_i[0, 8:], 1)

    den2_hi = _make_kA(N_ITEM)(shi, dhi, el2hi, er2hi, c_hi2, zeros16)
    alpha2_hi = _make_kB()(shi, dhi, el2hi, er2hi,
                           den2_hi[:N_ITEM], den2_hi[N_ITEM:], c_hi2)
    hi2_parts = _make_kD(N_FEAT, N_ITEM, 1, 1, halves=2)(
        shi, dhi, alpha2_hi, f2_hi,
        jnp.zeros((N_ITEM // (2 * NS), 32), jnp.float32))

    idx_tgt = jnp.nonzero(is_target, size=BS, fill_value=0)[0].astype(jnp.int32)
    tgt_parts = _make_tgt_gather()(hi2_parts, idx_tgt.reshape(NW, SUB))
    return _tc_final(rows_user, W_user, b_user.reshape(1, 32),
                     tgt_parts.reshape(2, BS, 32), b2_hi.reshape(1, 32))


# speedup vs baseline: 1.2404x; 1.2199x over previous
import functools

import jax
import jax.numpy as jnp
from jax import lax
from jax.experimental import pallas as pl
from jax.experimental.pallas import tpu as pltpu
from jax.experimental.pallas import tpu_sc as plsc

N_ITEM = 40000
N_FEAT = 10000
E = 400000
BS = 4096
EMB = 32
H1 = 8
OUT = 32

NC = 2
NS = 16
NW = NC * NS
SUB = 128
NSUB = 8
CHUNK = SUB * NSUB
E_PAD = ((E + CHUNK - 1) // CHUNK) * CHUNK
NCHUNKS = E_PAD // CHUNK

_mesh = plsc.VectorSubcoreMesh(core_axis_name="c", subcore_axis_name="s")
_sc_params = pltpu.CompilerParams(use_tc_tiling_on_sc=False,
                                  needs_layout_passes=False)


def _iota16():
    return lax.iota(jnp.int32, 16)


def _splat(x):
    return jnp.full((16,), x, jnp.int32)


def _vread(ref, i, off):
    return plsc.load_gather(ref, [_splat(i), _iota16() + off])


def _vwrite(ref, i, off, val):
    plsc.store_scatter(ref, [_splat(i), _iota16() + off], val)


def _edge_mask(gid):
    one = jnp.full((16,), 1.0, jnp.float32)
    zero = jnp.full((16,), 0.0, jnp.float32)
    return jnp.where(_splat(gid) < E, one, zero)


def _make_gather(n_rows_pad, table_rows):
    nloc = n_rows_pad // (NW * SUB)

    @functools.partial(
        pl.kernel, mesh=_mesh, compiler_params=_sc_params,
        out_type=jax.ShapeDtypeStruct((n_rows_pad, 32), jnp.float32),
        scratch_types=[
            pltpu.VMEM((SUB,), jnp.int32),
            pltpu.VMEM((SUB, 32), jnp.float32),
            pltpu.SemaphoreType.DMA,
        ],
    )
    def k(table_hbm, idx_hbm, out_hbm, idx_v, rows_v, sem):
        w = lax.axis_index("c") * NS + lax.axis_index("s")

        def body(j, carry):
            r = w * nloc + j
            pltpu.sync_copy(idx_hbm.at[r], idx_v)
            pltpu.async_copy(table_hbm.at[idx_v], rows_v, sem).wait()
            pltpu.sync_copy(rows_v, out_hbm.at[pl.ds(r * SUB, SUB)])
            return carry

        lax.fori_loop(0, nloc, body, 0)

    return k


def _make_kA(n_dst, halves=1):
    half = n_dst // halves
    rpt = half // NS

    @functools.partial(
        pl.kernel, mesh=_mesh, compiler_params=_sc_params,
        out_type=jax.ShapeDtypeStruct((NC * n_dst, 16), jnp.float32),
        scratch_types=[
            pltpu.VMEM((NSUB, SUB), jnp.int32),
            pltpu.VMEM((NSUB, SUB), jnp.int32),
            pltpu.VMEM((NSUB, SUB), jnp.int32),
            pltpu.VMEM((CHUNK, 16), jnp.float32),
            pltpu.VMEM((CHUNK, 16), jnp.float32),
            pltpu.VMEM((CHUNK, 16), jnp.float32),
            pltpu.VMEM((16,), jnp.float32),
            pltpu.VMEM_SHARED((half, 16), jnp.float32),
            pltpu.SemaphoreType.DMA,
            pltpu.SemaphoreType.DMA,
            pltpu.SemaphoreType.DMA,
        ],
    )
    def k(src_hbm, dst_hbm, el_hbm, er_hbm, c_hbm, zeros_hbm, den_out,
          src_v, dst_v, dst2_v, el_v, er_v, ee_v, c_v, acc, s_el, s_er, s_sc):
        cid = lax.axis_index("c")
        sid = lax.axis_index("s")
        w = cid * NS + sid
        pltpu.sync_copy(c_hbm, c_v)
        nch = (NCHUNKS - 1 - w) // NW + 1

        def pass_body(hh, carry0):
            lo = hh * half
            pltpu.sync_copy(zeros_hbm, acc.at[pl.ds(sid * rpt, rpt)])
            plsc.subcore_barrier()

            def chunk(j, carry):
                g = w + j * NW
                pltpu.sync_copy(src_hbm.at[pl.ds(g * NSUB, NSUB)], src_v)
                pltpu.sync_copy(dst_hbm.at[pl.ds(g * NSUB, NSUB)], dst_v)
                cps = []
                for sb in range(NSUB):
                    cps.append(pltpu.async_copy(
                        el_hbm.at[src_v.at[sb]],
                        el_v.at[pl.ds(sb * SUB, SUB)], s_el))
                    cps.append(pltpu.async_copy(
                        er_hbm.at[dst_v.at[sb]],
                        er_v.at[pl.ds(sb * SUB, SUB)], s_er))
                if halves > 1:
                    for sb in range(NSUB):
                        for m in range(NSUB):
                            ii = _iota16() + m * 16
                            d = plsc.load_gather(dst_v, [_splat(sb), ii]) - lo
                            d = jnp.clip(d, 0, half - 1)
                            plsc.store_scatter(dst2_v, [_splat(sb), ii], d)
                for cp in cps:
                    cp.wait()
                cvec = c_v[...]

                def edge(i, c2):
                    ev = _vread(el_v, i, 0) + _vread(er_v, i, 0)
                    ev = jnp.maximum(ev, 0.2 * ev)
                    msk = _edge_mask(g * CHUNK + i)
                    if halves > 1:
                        dv = plsc.load_gather(
                            dst_v, [_splat(i // SUB), _splat(i % SUB)])
                        one = jnp.full((16,), 1.0, jnp.float32)
                        zero = jnp.full((16,), 0.0, jnp.float32)
                        inr = jnp.where((dv >= lo) & (dv < lo + half),
                                        one, zero)
                        msk = msk * inr
                    ee = jnp.exp(ev - cvec) * msk
                    _vwrite(ee_v, i, 0, ee)
                    return c2

                lax.fori_loop(0, CHUNK, edge, 0, unroll=2)
                didx = dst2_v if halves > 1 else dst_v
                scs = []
                for sb in range(NSUB):
                    scs.append(pltpu.async_copy(
                        ee_v.at[pl.ds(sb * SUB, SUB)],
                        acc.at[didx.at[sb]], s_sc, add=True))
                for cp in scs:
                    cp.wait()
                return carry

            lax.fori_loop(0, nch, chunk, 0)
            plsc.subcore_barrier()
            pltpu.sync_copy(
                acc.at[pl.ds(sid * rpt, rpt)],
                den_out.at[pl.ds(cid * n_dst + lo + sid * rpt, rpt)])
            plsc.subcore_barrier()
            return carry0

        lax.fori_loop(0, halves, pass_body, 0)

    return k


def _make_kB():
    @functools.partial(
        pl.kernel, mesh=_mesh, compiler_params=_sc_params,
        out_type=jax.ShapeDtypeStruct((E_PAD, 16), jnp.float32),
        scratch_types=[
            pltpu.VMEM((NSUB, SUB), jnp.int32),
            pltpu.VMEM((NSUB, SUB), jnp.int32),
            pltpu.VMEM((CHUNK, 16), jnp.float32),
            pltpu.VMEM((CHUNK, 16), jnp.float32),
            pltpu.VMEM((CHUNK, 16), jnp.float32),
            pltpu.VMEM((CHUNK, 16), jnp.float32),
            pltpu.VMEM((CHUNK, 16), jnp.float32),
            pltpu.VMEM((16,), jnp.float32),
            pltpu.SemaphoreType.DMA,
            pltpu.SemaphoreType.DMA,
            pltpu.SemaphoreType.DMA,
            pltpu.SemaphoreType.DMA,
        ],
    )
    def k(src_hbm, dst_hbm, el_hbm, er_hbm, d0_hbm, d1_hbm, c_hbm, alpha_out,
          src_v, dst_v, el_v, er_v, d0_v, d1_v, a_v, c_v, s1, s2, s3, s4):
        cid = lax.axis_index("c")
        sid = lax.axis_index("s")
        w = cid * NS + sid
        pltpu.sync_copy(c_hbm, c_v)
        nch = (NCHUNKS - 1 - w) // NW + 1
        eps = jnp.full((16,), 1e-16, jnp.float32)

        def chunk(j, carry):
            g = w + j * NW
            pltpu.sync_copy(src_hbm.at[pl.ds(g * NSUB, NSUB)], src_v)
            pltpu.sync_copy(dst_hbm.at[pl.ds(g * NSUB, NSUB)], dst_v)
            cps = []
            for sb in range(NSUB):
                cps.append(pltpu.async_copy(
                    el_hbm.at[src_v.at[sb]],
                    el_v.at[pl.ds(sb * SUB, SUB)], s1))
                cps.append(pltpu.async_copy(
                    er_hbm.at[dst_v.at[sb]],
                    er_v.at[pl.ds(sb * SUB, SUB)], s2))
                cps.append(pltpu.async_copy(
                    d0_hbm.at[dst_v.at[sb]],
                    d0_v.at[pl.ds(sb * SUB, SUB)], s3))
                cps.append(pltpu.async_copy(
                    d1_hbm.at[dst_v.at[sb]],
                    d1_v.at[pl.ds(sb * SUB, SUB)], s4))
            for cp in cps:
                cp.wait()
            cvec = c_v[...]

            def edge(i, c2):
                ev = _vread(el_v, i, 0) + _vread(er_v, i, 0)
                ev = jnp.maximum(ev, 0.2 * ev)
                ee = jnp.exp(ev - cvec)
                den = _vread(d0_v, i, 0) + _vread(d1_v, i, 0)
                alpha = ee / jnp.maximum(den, eps)
                _vwrite(a_v, i, 0, alpha)
                return c2

            lax.fori_loop(0, CHUNK, edge, 0, unroll=2)
            pltpu.sync_copy(a_v, alpha_out.at[pl.ds(g * CHUNK, CHUNK)])
            return carry

        lax.fori_loop(0, nch, chunk, 0)

    return k


def _make_kD(n_src, n_dst, H, G, halves=1):
    NG = H // G
    W = G * 32
    half = n_dst // halves
    rpt = half // NS
    NBUF = 4

    @functools.partial(
        pl.kernel, mesh=_mesh, compiler_params=_sc_params,
        out_type=jax.ShapeDtypeStruct((NC * NG * n_dst, W), jnp.float32),
        scratch_types=[
            pltpu.VMEM((NSUB, SUB), jnp.int32),
            pltpu.VMEM((NSUB, SUB), jnp.int32),
            pltpu.VMEM((NSUB, SUB), jnp.int32),
            pltpu.VMEM((NSUB, SUB), jnp.int32),
            pltpu.VMEM((CHUNK, 16), jnp.float32),
            pltpu.VMEM((NBUF * SUB, W), jnp.float32),
            pltpu.VMEM_SHARED((half, W), jnp.float32),
            pltpu.SemaphoreType.DMA,
            pltpu.SemaphoreType.DMA,
        ],
    )
    def k(edges_hbm, alpha_hbm, fsrc_hbm, zeros_hbm, out_hbm,
          raw_v, dst_v, idx2_v, dst2_v, a_v, rows_v, acc, s_g, s_s):
        cid = lax.axis_index("c")
        sid = lax.axis_index("s")
        w = cid * NS + sid
        nch = (NCHUNKS - 1 - w) // NW + 1

        def pass_body(t, carry0):
            grp = t // halves
            hh = t % halves
            if True:
                lo = hh * half
                pltpu.sync_copy(zeros_hbm, acc.at[pl.ds(sid * rpt, rpt)])
                plsc.subcore_barrier()

                def chunk(j, carry):
                    g = w + j * NW
                    pltpu.sync_copy(edges_hbm.at[pl.ds(g * NSUB, NSUB)], raw_v)
                    pltpu.sync_copy(alpha_hbm.at[pl.ds(g * CHUNK, CHUNK)], a_v)
                    mask16 = jnp.full((16,), 0xFFFF, jnp.int32)
                    for sb in range(NSUB):
                        for m in range(NSUB):
                            ii = _iota16() + m * 16
                            v = plsc.load_gather(raw_v, [_splat(sb), ii])
                            plsc.store_scatter(idx2_v, [_splat(sb), ii],
                                               (v & mask16) + grp * n_src)
                            d = lax.shift_right_logical(v, 16)
                            plsc.store_scatter(dst_v, [_splat(sb), ii], d)
                            if halves > 1:
                                d = jnp.clip(d - lo, 0, half - 1)
                                plsc.store_scatter(dst2_v, [_splat(sb), ii], d)
                    didx = dst2_v if halves > 1 else dst_v
                    for r in range(NSUB // NBUF):
                        cps = []
                        for b in range(NBUF):
                            sb = r * NBUF + b
                            cps.append(pltpu.async_copy(
                                fsrc_hbm.at[idx2_v.at[sb]],
                                rows_v.at[pl.ds(b * SUB, SUB)], s_g))
                        for cp in cps:
                            cp.wait()
                        scs = []
                        for b in range(NBUF):
                            sb = r * NBUF + b
                            base = g * CHUNK + sb * SUB

                            def edge(i, c2):
                                msk = _edge_mask(base + i)
                                if halves > 1:
                                    dv = plsc.load_gather(
                                        dst_v, [_splat(sb), _splat(i)])
                                    one = jnp.full((16,), 1.0, jnp.float32)
                                    zero = jnp.full((16,), 0.0, jnp.float32)
                                    inr = jnp.where(
                                        (dv >= lo) & (dv < lo + half),
                                        one, zero)
                                    msk = msk * inr
                                row = b * SUB + i
                                for jj in range(G):
                                    av = plsc.load_gather(
                                        a_v, [_splat(sb * SUB + i),
                                              _splat(grp * G + jj)]) * msk
                                    _vwrite(rows_v, row, 32 * jj,
                                            _vread(rows_v, row, 32 * jj) * av)
                                    _vwrite(rows_v, row, 32 * jj + 16,
                                            _vread(rows_v, row, 32 * jj + 16) * av)
                                return c2

                            lax.fori_loop(0, SUB, edge, 0, unroll=2)
                            scs.append(pltpu.async_copy(
                                rows_v.at[pl.ds(b * SUB, SUB)],
                                acc.at[didx.at[sb]], s_s, add=True))
                        for cp in scs:
                            cp.wait()
                    return carry

                lax.fori_loop(0, nch, chunk, 0)
                plsc.subcore_barrier()
                pltpu.sync_copy(
                    acc.at[pl.ds(sid * rpt, rpt)],
                    out_hbm.at[pl.ds((cid * NG + grp) * n_dst + lo + sid * rpt,
                                     rpt)])
                plsc.subcore_barrier()
            return carry0

        lax.fori_loop(0, NG * halves, pass_body, 0)

    return k


def _make_tgt_gather():
    @functools.partial(
        pl.kernel, mesh=_mesh, compiler_params=_sc_params,
        out_type=jax.ShapeDtypeStruct((NC * BS, 32), jnp.float32),
        scratch_types=[
            pltpu.VMEM((SUB,), jnp.int32),
            pltpu.VMEM((SUB,), jnp.int32),
            pltpu.VMEM((SUB, 32), jnp.float32),
            pltpu.SemaphoreType.DMA,
        ],
    )
    def k(parts_hbm, idx_hbm, out_hbm, idx_v, idx2_v, rows_v, sem):
        w = lax.axis_index("c") * NS + lax.axis_index("s")
        pltpu.sync_copy(idx_hbm.at[w], idx_v)
        pltpu.async_copy(parts_hbm.at[idx_v], rows_v, sem).wait()
        pltpu.sync_copy(rows_v, out_hbm.at[pl.ds(w * SUB, SUB)])
        for m in range(NSUB):
            v = plsc.load_gather(idx_v, [_iota16() + m * 16])
            plsc.store_scatter(idx2_v, [_iota16() + m * 16], v + N_ITEM)
        pltpu.async_copy(parts_hbm.at[idx2_v], rows_v, sem).wait()
        pltpu.sync_copy(rows_v, out_hbm.at[pl.ds(BS + w * SUB, SUB)])

    return k


def _tc_item_l1(rows_id, rows_rat, ts, W_item, b_item, W1bt, al1bt, W1hi, ar1hi):
    BN = 2000
    grid = N_ITEM // BN

    def body(id_ref, rat_ref, ts_ref, Wi_ref, bi_ref, Wbt_ref, albt_ref,
             Whi_ref, arhi_ref, f_ref, el_ref, er_ref, mx_ref):
        x = jnp.concatenate([id_ref[...], rat_ref[...], ts_ref[...]], axis=1)
        emb = x @ Wi_ref[...] + bi_ref[...]
        f = emb @ Wbt_ref[...]
        gmat = emb @ Whi_ref[...]
        els = []
        ers = []
        for h in range(H1):
            fh = f[:, 32 * h:32 * h + 32]
            f_ref[h] = fh
            els.append(jnp.sum(fh * albt_ref[h][None, :], axis=1, keepdims=True))
            ers.append(jnp.sum(gmat[:, 32 * h:32 * h + 32] * arhi_ref[h][None, :],
                               axis=1, keepdims=True))
        z = jnp.zeros((BN, 8), jnp.float32)
        el16 = jnp.concatenate(els + [z], axis=1)
        er16 = jnp.concatenate(ers + [z], axis=1)
        el_ref[...] = el16
        er_ref[...] = er16
        m = jnp.concatenate([jnp.max(el16[:, :8], axis=0),
                             jnp.max(er16[:, :8], axis=0)])[None, :]

        @pl.when(pl.program_id(0) == 0)
        def _():
            mx_ref[...] = jnp.full((1, 16), -1e30, jnp.float32)

        mx_ref[...] = jnp.maximum(mx_ref[...], m)

    return pl.pallas_call(
        body,
        grid=(grid,),
        in_specs=[
            pl.BlockSpec((BN, 32), lambda i: (i, 0)),
            pl.BlockSpec((BN, 32), lambda i: (i, 0)),
            pl.BlockSpec((BN, 32), lambda i: (i, 0)),
            pl.BlockSpec((96, 32), lambda i: (0, 0)),
            pl.BlockSpec((1, 32), lambda i: (0, 0)),
            pl.BlockSpec((32, 256), lambda i: (0, 0)),
            pl.BlockSpec((8, 32), lambda i: (0, 0)),
            pl.BlockSpec((32, 256), lambda i: (0, 0)),
            pl.BlockSpec((8, 32), lambda i: (0, 0)),
        ],
        out_specs=[
            pl.BlockSpec((8, BN, 32), lambda i: (0, i, 0)),
            pl.BlockSpec((BN, 16), lambda i: (i, 0)),
            pl.BlockSpec((BN, 16), lambda i: (i, 0)),
            pl.BlockSpec((1, 16), lambda i: (0, 0)),
        ],
        out_shape=[
            jax.ShapeDtypeStruct((8, N_ITEM, 32), jnp.float32),
            jax.ShapeDtypeStruct((N_ITEM, 16), jnp.float32),
            jax.ShapeDtypeStruct((N_ITEM, 16), jnp.float32),
            jax.ShapeDtypeStruct((1, 16), jnp.float32),
        ],
    )(rows_id, rows_rat, ts, W_item, b_item, W1bt, al1bt, W1hi, ar1hi)


def _tc_feat_l1(rows_feat, W_feat, b_feat, W1hi, al1hi, W1bt, ar1bt):
    BN = 2000
    grid = N_FEAT // BN

    def body(rf_ref, Wf_ref, bf_ref, Whi_ref, alhi_ref, Wbt_ref, arbt_ref,
             f_ref, el_ref, er_ref, mx_ref):
        emb = rf_ref[...] @ Wf_ref[...] + bf_ref[...]
        f = emb @ Whi_ref[...]
        gmat = emb @ Wbt_ref[...]
        els = []
        ers = []
        for h in range(H1):
            fh = f[:, 32 * h:32 * h + 32]
            f_ref[h] = fh
            els.append(jnp.sum(fh * alhi_ref[h][None, :], axis=1, keepdims=True))
            ers.append(jnp.sum(gmat[:, 32 * h:32 * h + 32] * arbt_ref[h][None, :],
                               axis=1, keepdims=True))
        z = jnp.zeros((BN, 8), jnp.float32)
        el16 = jnp.concatenate(els + [z], axis=1)
        er16 = jnp.concatenate(ers + [z], axis=1)
        el_ref[...] = el16
        er_ref[...] = er16
        m = jnp.concatenate([jnp.max(el16[:, :8], axis=0),
                             jnp.max(er16[:, :8], axis=0)])[None, :]

        @pl.when(pl.program_id(0) == 0)
        def _():
            mx_ref[...] = jnp.full((1, 16), -1e30, jnp.float32)

        mx_ref[...] = jnp.maximum(mx_ref[...], m)

    return pl.pallas_call(
        body,
        grid=(grid,),
        in_specs=[
            pl.BlockSpec((BN, 32), lambda i: (i, 0)),
            pl.BlockSpec((32, 32), lambda i: (0, 0)),
            pl.BlockSpec((1, 32), lambda i: (0, 0)),
            pl.BlockSpec((32, 256), lambda i: (0, 0)),
            pl.BlockSpec((8, 32), lambda i: (0, 0)),
            pl.BlockSpec((32, 256), lambda i: (0, 0)),
            pl.BlockSpec((8, 32), lambda i: (0, 0)),
        ],
        out_specs=[
            pl.BlockSpec((8, BN, 32), lambda i: (0, i, 0)),
            pl.BlockSpec((BN, 16), lambda i: (i, 0)),
            pl.BlockSpec((BN, 16), lambda i: (i, 0)),
            pl.BlockSpec((1, 16), lambda i: (0, 0)),
        ],
        out_shape=[
            jax.ShapeDtypeStruct((H1, N_FEAT, 32), jnp.float32),
            jax.ShapeDtypeStruct((N_FEAT, 16), jnp.float32),
            jax.ShapeDtypeStruct((N_FEAT, 16), jnp.float32),
            jax.ShapeDtypeStruct((1, 16), jnp.float32),
        ],
    )(rows_feat, W_feat, b_feat, W1hi, al1hi, W1bt, ar1bt)


def _tc_l2_prep(parts, b1, W2a, al2a, W2b, ar2b, n, NG, G):
    BN = 2000
    grid = n // BN
    W = G * 32

    def body(p_ref, b1_ref, Wa_ref, ala_ref, Wb_ref, arb_ref,
             f_ref, el_ref, er_ref, mx_ref):
        cols = []
        for h in range(H1):
            grp, jj = divmod(h, G)
            cols.append(p_ref[0, grp, :, 32 * jj:32 * jj + 32]
                        + p_ref[1, grp, :, 32 * jj:32 * jj + 32])
        hmat = jnp.concatenate(cols, axis=1) + b1_ref[...]
        f2 = hmat @ Wa_ref[...]
        g2 = hmat @ Wb_ref[...]
        f_ref[...] = f2
        el = jnp.sum(f2 * ala_ref[0][None, :], axis=1, keepdims=True)
        er = jnp.sum(g2 * arb_ref[0][None, :], axis=1, keepdims=True)
        z = jnp.zeros((BN, 15), jnp.float32)
        el16 = jnp.concatenate([el, z], axis=1)
        er16 = jnp.concatenate([er, z], axis=1)
        el_ref[...] = el16
        er_ref[...] = er16
        m = jnp.concatenate(
            [jnp.max(el16[:, :8], axis=0), jnp.max(er16[:, :8], axis=0)])[None, :]

        @pl.when(pl.program_id(0) == 0)
        def _():
            mx_ref[...] = jnp.full((1, 16), -1e30, jnp.float32)

        mx_ref[...] = jnp.maximum(mx_ref[...], m)

    return pl.pallas_call(
        body,
        grid=(grid,),
        in_specs=[
            pl.BlockSpec((2, NG, BN, W), lambda i: (0, 0, i, 0)),
            pl.BlockSpec((1, 256), lambda i: (0, 0)),
            pl.BlockSpec((256, 32), lambda i: (0, 0)),
            pl.BlockSpec((1, 32), lambda i: (0, 0)),
            pl.BlockSpec((256, 32), lambda i: (0, 0)),
            pl.BlockSpec((1, 32), lambda i: (0, 0)),
        ],
        out_specs=[
            pl.BlockSpec((BN, 32), lambda i: (i, 0)),
            pl.BlockSpec((BN, 16), lambda i: (i, 0)),
            pl.BlockSpec((BN, 16), lambda i: (i, 0)),
            pl.BlockSpec((1, 16), lambda i: (0, 0)),
        ],
        out_shape=[
            jax.ShapeDtypeStruct((n, 32), jnp.float32),
            jax.ShapeDtypeStruct((n, 16), jnp.float32),
            jax.ShapeDtypeStruct((n, 16), jnp.float32),
            jax.ShapeDtypeStruct((1, 16), jnp.float32),
        ],
    )(parts, b1, W2a, al2a, W2b, ar2b)


def _tc_final(rows_user, W_user, b_user, tgt_parts, b2hi):
    def body(ru_ref, Wu_ref, bu_ref, tp_ref, b2_ref, o_ref):
        u = ru_ref[...] @ Wu_ref[...] + bu_ref[...]
        t = tp_ref[0] + tp_ref[1] + b2_ref[...]
        num = jnp.sum(u * t, axis=1)
        nu = jnp.maximum(jnp.sqrt(jnp.sum(u * u, axis=1)), 1e-8)
        nt = jnp.maximum(jnp.sqrt(jnp.sum(t * t, axis=1)), 1e-8)
        o_ref[...] = jax.nn.sigmoid(num / (nu * nt))

    return pl.pallas_call(
        body,
        out_shape=jax.ShapeDtypeStruct((BS,), jnp.float32),
    )(rows_user, W_user, b_user, tgt_parts, b2hi)


def _pad_edges(x):
    return jnp.pad(x, (0, E_PAD - E)).reshape(E_PAD // SUB, SUB).astype(jnp.int32)


def _c16(m, nh):
    c = jax.nn.leaky_relu(m, 0.2)
    return jnp.concatenate([c[:nh], jnp.full((16 - nh,), 100.0, jnp.float32)])


def kernel(user_feat, item_id, item_rating, feat_id, is_target,
           edge_bt_src, edge_bt_dst, edge_hi_src, edge_hi_dst,
           ts_emb, emb_table, W_user, b_user, W_item, b_item, W_feat, b_feat,
           W1_bt, al1_bt, ar1_bt, b1_bt, W1_hi, al1_hi, ar1_hi, b1_hi,
           W2_bt, al2_bt, ar2_bt, b2_bt, W2_hi, al2_hi, ar2_hi, b2_hi):
    zeros16 = jnp.zeros((2500, 16), jnp.float32)

    n_all = N_ITEM + N_FEAT + BS + N_ITEM
    n_all_pad = ((n_all + NW * SUB - 1) // (NW * SUB)) * (NW * SUB)
    idx_all = jnp.concatenate([
        item_id.astype(jnp.int32), feat_id.astype(jnp.int32),
        user_feat[:, 0].astype(jnp.int32), item_rating.astype(jnp.int32),
        jnp.zeros((n_all_pad - n_all,), jnp.int32)]).reshape(n_all_pad // SUB, SUB)
    rows = _make_gather(n_all_pad, emb_table.shape[0])(emb_table, idx_all)
    rows_id = rows[:N_ITEM]
    rows_feat = rows[N_ITEM:N_ITEM + N_FEAT]
    rows_user = rows[N_ITEM + N_FEAT:N_ITEM + N_FEAT + BS]
    rows_rat = rows[N_ITEM + N_FEAT + BS:n_all]

    f1_bt, el1bt, er1hi, mx_i = _tc_item_l1(
        rows_id, rows_rat, ts_emb, W_item, b_item.reshape(1, 32),
        W1_bt, al1_bt, W1_hi, ar1_hi)
    f1_hi, el1hi, er1bt, mx_f = _tc_feat_l1(
        rows_feat, W_feat, b_feat.reshape(1, 32), W1_hi, al1_hi, W1_bt, ar1_bt)

    c_bt1 = _c16(mx_i[0, :8] + mx_f[0, 8:], 8)
    c_hi1 = _c16(mx_f[0, :8] + mx_i[0, 8:], 8)

    sbt = _pad_edges(edge_bt_src)
    dbt = _pad_edges(edge_bt_dst)
    shi = _pad_edges(edge_hi_src)
    dhi = _pad_edges(edge_hi_dst)

    zeros_a = jnp.zeros((N_ITEM // (2 * NS), 16), jnp.float32)
    den_bt = _make_kA(N_FEAT)(sbt, dbt, el1bt, er1bt, c_bt1,
                              jnp.zeros((N_FEAT // NS, 16), jnp.float32))
    den_hi = _make_kA(N_ITEM, halves=2)(shi, dhi, el1hi, er1hi, c_hi1, zeros_a)
    alpha_bt = _make_kB()(sbt, dbt, el1bt, er1bt,
                          den_bt[:N_FEAT], den_bt[N_FEAT:], c_bt1)
    alpha_hi = _make_kB()(shi, dhi, el1hi, er1hi,
                          den_hi[:N_ITEM], den_hi[N_ITEM:], c_hi1)
    ebt = jnp.bitwise_or(jnp.left_shift(dbt, 16), sbt)
    ehi = jnp.bitwise_or(jnp.left_shift(dhi, 16), shi)
    bt_parts = _make_kD(N_ITEM, N_FEAT, H1, 1)(
        ebt, alpha_bt, f1_bt.reshape(H1 * N_ITEM, 32),
        jnp.zeros((N_FEAT // NS, 32), jnp.float32))
    hi_parts = _make_kD(N_FEAT, N_ITEM, H1, 1)(
        ehi, alpha_hi, f1_hi.reshape(H1 * N_FEAT, 32),
        jnp.zeros((N_ITEM // NS, 32), jnp.float32))

    f2_bt, el2bt, er2hi, mx2_i = _tc_l2_prep(
        hi_parts.reshape(2, H1, N_ITEM, 32), b1_hi.reshape(1, 256),
        W2_bt, al2_bt, W2_hi, ar2_hi, N_ITEM, H1, 1)
    f2_hi, el2hi, er2bt, mx2_f = _tc_l2_prep(
        bt_parts.reshape(2, H1, N_FEAT, 32), b1_bt.reshape(1, 256),
        W2_hi, al2_hi, W2_bt, ar2_bt, N_FEAT, H1, 1)

    c_hi2 = _c16(mx2_f[0, :8] + mx2_i[0, 8:], 1)

    den2_hi = _make_kA(N_ITEM, halves=2)(shi, dhi, el2hi, er2hi, c_hi2,
                                         zeros_a)
    alpha2_hi = _make_kB()(shi, dhi, el2hi, er2hi,
                           den2_hi[:N_ITEM], den2_hi[N_ITEM:], c_hi2)
    hi2_parts = _make_kD(N_FEAT, N_ITEM, 1, 1)(
        ehi, alpha2_hi, f2_hi,
        jnp.zeros((N_ITEM // NS, 32), jnp.float32))

    idx_tgt = jnp.nonzero(is_target, size=BS, fill_value=0)[0].astype(jnp.int32)
    tgt_parts = _make_tgt_gather()(hi2_parts, idx_tgt.reshape(NW, SUB))
    return _tc_final(rows_user, W_user, b_user.reshape(1, 32),
                     tgt_parts.reshape(2, BS, 32), b2_hi.reshape(1, 32))


# speedup vs baseline: 1.3539x; 1.0915x over previous
import functools

import jax
import jax.numpy as jnp
from jax import lax
from jax.experimental import pallas as pl
from jax.experimental.pallas import tpu as pltpu
from jax.experimental.pallas import tpu_sc as plsc

N_ITEM = 40000
N_FEAT = 10000
E = 400000
BS = 4096
EMB = 32
H1 = 8
OUT = 32

NC = 2
NS = 16
NW = NC * NS
SUB = 128
NSUB = 8
CHUNK = SUB * NSUB
E_PAD = ((E + CHUNK - 1) // CHUNK) * CHUNK
NCHUNKS = E_PAD // CHUNK

_mesh = plsc.VectorSubcoreMesh(core_axis_name="c", subcore_axis_name="s")
_sc_params = pltpu.CompilerParams(use_tc_tiling_on_sc=False,
                                  needs_layout_passes=False)


def _iota16():
    return lax.iota(jnp.int32, 16)


def _splat(x):
    return jnp.full((16,), x, jnp.int32)


def _vread(ref, i, off):
    return plsc.load_gather(ref, [_splat(i), _iota16() + off])


def _vwrite(ref, i, off, val):
    plsc.store_scatter(ref, [_splat(i), _iota16() + off], val)


def _edge_mask(gid):
    one = jnp.full((16,), 1.0, jnp.float32)
    zero = jnp.full((16,), 0.0, jnp.float32)
    return jnp.where(_splat(gid) < E, one, zero)


def _make_gather(n_rows_pad, table_rows):
    nloc = n_rows_pad // (NW * SUB)

    @functools.partial(
        pl.kernel, mesh=_mesh, compiler_params=_sc_params,
        out_type=jax.ShapeDtypeStruct((n_rows_pad, 32), jnp.float32),
        scratch_types=[
            pltpu.VMEM((SUB,), jnp.int32),
            pltpu.VMEM((SUB, 32), jnp.float32),
            pltpu.SemaphoreType.DMA,
        ],
    )
    def k(table_hbm, idx_hbm, out_hbm, idx_v, rows_v, sem):
        w = lax.axis_index("c") * NS + lax.axis_index("s")

        def body(j, carry):
            r = w * nloc + j
            pltpu.sync_copy(idx_hbm.at[r], idx_v)
            pltpu.async_copy(table_hbm.at[idx_v], rows_v, sem).wait()
            pltpu.sync_copy(rows_v, out_hbm.at[pl.ds(r * SUB, SUB)])
            return carry

        lax.fori_loop(0, nloc, body, 0)

    return k


def _make_kA(n_dst, halves=1):
    half = n_dst // halves
    rpt = half // NS

    @functools.partial(
        pl.kernel, mesh=_mesh, compiler_params=_sc_params,
        out_type=jax.ShapeDtypeStruct((NC * n_dst, 16), jnp.float32),
        scratch_types=[
            pltpu.VMEM((NSUB, SUB), jnp.int32),
            pltpu.VMEM((NSUB, SUB), jnp.int32),
            pltpu.VMEM((NSUB, SUB), jnp.int32),
            pltpu.VMEM((CHUNK, 16), jnp.float32),
            pltpu.VMEM((CHUNK, 16), jnp.float32),
            pltpu.VMEM((CHUNK, 16), jnp.float32),
            pltpu.VMEM((16,), jnp.float32),
            pltpu.VMEM_SHARED((half, 16), jnp.float32),
            pltpu.SemaphoreType.DMA,
            pltpu.SemaphoreType.DMA,
            pltpu.SemaphoreType.DMA,
        ],
    )
    def k(src_hbm, dst_hbm, el_hbm, er_hbm, c_hbm, zeros_hbm, den_out,
          src_v, dst_v, dst2_v, el_v, er_v, ee_v, c_v, acc, s_el, s_er, s_sc):
        cid = lax.axis_index("c")
        sid = lax.axis_index("s")
        w = cid * NS + sid
        pltpu.sync_copy(c_hbm, c_v)
        nch = (NCHUNKS - 1 - w) // NW + 1

        def pass_body(hh, carry0):
            lo = hh * half
            pltpu.sync_copy(zeros_hbm, acc.at[pl.ds(sid * rpt, rpt)])
            plsc.subcore_barrier()

            def chunk(j, carry):
                g = w + j * NW
                pltpu.sync_copy(src_hbm.at[pl.ds(g * NSUB, NSUB)], src_v)
                pltpu.sync_copy(dst_hbm.at[pl.ds(g * NSUB, NSUB)], dst_v)
                cps = []
                for sb in range(NSUB):
                    cps.append(pltpu.async_copy(
                        el_hbm.at[src_v.at[sb]],
                        el_v.at[pl.ds(sb * SUB, SUB)], s_el))
                    cps.append(pltpu.async_copy(
                        er_hbm.at[dst_v.at[sb]],
                        er_v.at[pl.ds(sb * SUB, SUB)], s_er))
                if halves > 1:
                    for sb in range(NSUB):
                        for m in range(NSUB):
                            ii = _iota16() + m * 16
                            d = plsc.load_gather(dst_v, [_splat(sb), ii]) - lo
                            d = jnp.clip(d, 0, half - 1)
                            plsc.store_scatter(dst2_v, [_splat(sb), ii], d)
                for cp in cps:
                    cp.wait()
                cvec = c_v[...]

                def edge(i, c2):
                    ev = _vread(el_v, i, 0) + _vread(er_v, i, 0)
                    ev = jnp.maximum(ev, 0.2 * ev)
                    msk = _edge_mask(g * CHUNK + i)
                    if halves > 1:
                        dv = plsc.load_gather(
                            dst_v, [_splat(i // SUB), _splat(i % SUB)])
                        one = jnp.full((16,), 1.0, jnp.float32)
                        zero = jnp.full((16,), 0.0, jnp.float32)
                        inr = jnp.where((dv >= lo) & (dv < lo + half),
                                        one, zero)
                        msk = msk * inr
                    ee = jnp.exp(ev - cvec) * msk
                    _vwrite(ee_v, i, 0, ee)
                    return c2

                lax.fori_loop(0, CHUNK, edge, 0, unroll=2)
                didx = dst2_v if halves > 1 else dst_v
                scs = []
                for sb in range(NSUB):
                    scs.append(pltpu.async_copy(
                        ee_v.at[pl.ds(sb * SUB, SUB)],
                        acc.at[didx.at[sb]], s_sc, add=True))
                for cp in scs:
                    cp.wait()
                return carry

            lax.fori_loop(0, nch, chunk, 0)
            plsc.subcore_barrier()
            pltpu.sync_copy(
                acc.at[pl.ds(sid * rpt, rpt)],
                den_out.at[pl.ds(cid * n_dst + lo + sid * rpt, rpt)])
            plsc.subcore_barrier()
            return carry0

        lax.fori_loop(0, halves, pass_body, 0)

    return k


def _make_kB():
    @functools.partial(
        pl.kernel, mesh=_mesh, compiler_params=_sc_params,
        out_type=jax.ShapeDtypeStruct((E_PAD, 16), jnp.float32),
        scratch_types=[
            pltpu.VMEM((NSUB, SUB), jnp.int32),
            pltpu.VMEM((NSUB, SUB), jnp.int32),
            pltpu.VMEM((CHUNK, 16), jnp.float32),
            pltpu.VMEM((CHUNK, 16), jnp.float32),
            pltpu.VMEM((CHUNK, 16), jnp.float32),
            pltpu.VMEM((CHUNK, 16), jnp.float32),
            pltpu.VMEM((CHUNK, 16), jnp.float32),
            pltpu.VMEM((16,), jnp.float32),
            pltpu.SemaphoreType.DMA,
            pltpu.SemaphoreType.DMA,
            pltpu.SemaphoreType.DMA,
            pltpu.SemaphoreType.DMA,
        ],
    )
    def k(src_hbm, dst_hbm, el_hbm, er_hbm, d0_hbm, d1_hbm, c_hbm, alpha_out,
          src_v, dst_v, el_v, er_v, d0_v, d1_v, a_v, c_v, s1, s2, s3, s4):
        cid = lax.axis_index("c")
        sid = lax.axis_index("s")
        w = cid * NS + sid
        pltpu.sync_copy(c_hbm, c_v)
        nch = (NCHUNKS - 1 - w) // NW + 1
        eps = jnp.full((16,), 1e-16, jnp.float32)

        def chunk(j, carry):
            g = w + j * NW
            pltpu.sync_copy(src_hbm.at[pl.ds(g * NSUB, NSUB)], src_v)
            pltpu.sync_copy(dst_hbm.at[pl.ds(g * NSUB, NSUB)], dst_v)
            cps = []
            for sb in range(NSUB):
                cps.append(pltpu.async_copy(
                    el_hbm.at[src_v.at[sb]],
                    el_v.at[pl.ds(sb * SUB, SUB)], s1))
                cps.append(pltpu.async_copy(
                    er_hbm.at[dst_v.at[sb]],
                    er_v.at[pl.ds(sb * SUB, SUB)], s2))
                cps.append(pltpu.async_copy(
                    d0_hbm.at[dst_v.at[sb]],
                    d0_v.at[pl.ds(sb * SUB, SUB)], s3))
                cps.append(pltpu.async_copy(
                    d1_hbm.at[dst_v.at[sb]],
                    d1_v.at[pl.ds(sb * SUB, SUB)], s4))
            for cp in cps:
                cp.wait()
            cvec = c_v[...]

            def edge(i, c2):
                ev = _vread(el_v, i, 0) + _vread(er_v, i, 0)
                ev = jnp.maximum(ev, 0.2 * ev)
                ee = jnp.exp(ev - cvec)
                den = _vread(d0_v, i, 0) + _vread(d1_v, i, 0)
                alpha = ee / jnp.maximum(den, eps)
                _vwrite(a_v, i, 0, alpha)
                return c2

            lax.fori_loop(0, CHUNK, edge, 0, unroll=2)
            pltpu.sync_copy(a_v, alpha_out.at[pl.ds(g * CHUNK, CHUNK)])
            return carry

        lax.fori_loop(0, nch, chunk, 0)

    return k


def _make_kD(n_src, n_dst, H, G, halves=1):
    NG = H // G
    W = G * 32
    half = n_dst // halves
    rpt = half // NS
    NBUF = 4

    @functools.partial(
        pl.kernel, mesh=_mesh, compiler_params=_sc_params,
        out_type=jax.ShapeDtypeStruct((NC * NG * n_dst, W), jnp.float32),
        scratch_types=[
            pltpu.VMEM((NSUB, SUB), jnp.int32),
            pltpu.VMEM((NSUB, SUB), jnp.int32),
            pltpu.VMEM((NSUB, SUB), jnp.int32),
            pltpu.VMEM((NSUB, SUB), jnp.int32),
            pltpu.VMEM((CHUNK, 16), jnp.float32),
            pltpu.VMEM((NBUF * SUB, W), jnp.float32),
            pltpu.VMEM_SHARED((half, W), jnp.float32),
            pltpu.SemaphoreType.DMA,
            pltpu.SemaphoreType.DMA,
        ],
    )
    def k(edges_hbm, alpha_hbm, fsrc_hbm, zeros_hbm, out_hbm,
          raw_v, dst_v, idx2_v, dst2_v, a_v, rows_v, acc, s_g, s_s):
        cid = lax.axis_index("c")
        sid = lax.axis_index("s")
        w = cid * NS + sid
        nch = (NCHUNKS - 1 - w) // NW + 1

        def pass_body(t, carry0):
            grp = t // halves
            hh = t % halves
            if True:
                lo = hh * half
                pltpu.sync_copy(zeros_hbm, acc.at[pl.ds(sid * rpt, rpt)])
                plsc.subcore_barrier()

                def chunk(j, carry):
                    g = w + j * NW
                    pltpu.sync_copy(edges_hbm.at[pl.ds(g * NSUB, NSUB)], raw_v)
                    pltpu.sync_copy(alpha_hbm.at[pl.ds(g * CHUNK, CHUNK)], a_v)
                    mask16 = jnp.full((16,), 0xFFFF, jnp.int32)
                    for sb in range(NSUB):
                        for m in range(NSUB):
                            ii = _iota16() + m * 16
                            v = plsc.load_gather(raw_v, [_splat(sb), ii])
                            plsc.store_scatter(idx2_v, [_splat(sb), ii],
                                               (v & mask16) + grp * n_src)
                            d = lax.shift_right_logical(v, 16)
                            plsc.store_scatter(dst_v, [_splat(sb), ii], d)
                            if halves > 1:
                                d = jnp.clip(d - lo, 0, half - 1)
                                plsc.store_scatter(dst2_v, [_splat(sb), ii], d)
                    didx = dst2_v if halves > 1 else dst_v
                    for r in range(NSUB // NBUF):
                        cps = []
                        for b in range(NBUF):
                            sb = r * NBUF + b
                            cps.append(pltpu.async_copy(
                                fsrc_hbm.at[idx2_v.at[sb]],
                                rows_v.at[pl.ds(b * SUB, SUB)], s_g))
                        for cp in cps:
                            cp.wait()
                        scs = []
                        for b in range(NBUF):
                            sb = r * NBUF + b
                            base = g * CHUNK + sb * SUB

                            def edge(i, c2):
                                msk = _edge_mask(base + i)
                                if halves > 1:
                                    dv = plsc.load_gather(
                                        dst_v, [_splat(sb), _splat(i)])
                                    one = jnp.full((16,), 1.0, jnp.float32)
                                    zero = jnp.full((16,), 0.0, jnp.float32)
                                    inr = jnp.where(
                                        (dv >= lo) & (dv < lo + half),
                                        one, zero)
                                    msk = msk * inr
                                row = b * SUB + i
                                for jj in range(G):
                                    av = plsc.load_gather(
                                        a_v, [_splat(sb * SUB + i),
                                              _splat(grp * G + jj)]) * msk
                                    _vwrite(rows_v, row, 32 * jj,
                                            _vread(rows_v, row, 32 * jj) * av)
                                    _vwrite(rows_v, row, 32 * jj + 16,
                                            _vread(rows_v, row, 32 * jj + 16) * av)
                                return c2

                            lax.fori_loop(0, SUB, edge, 0, unroll=2)
                            scs.append(pltpu.async_copy(
                                rows_v.at[pl.ds(b * SUB, SUB)],
                                acc.at[didx.at[sb]], s_s, add=True))
                        for cp in scs:
                            cp.wait()
                    return carry

                lax.fori_loop(0, nch, chunk, 0)
                plsc.subcore_barrier()
                pltpu.sync_copy(
                    acc.at[pl.ds(sid * rpt, rpt)],
                    out_hbm.at[pl.ds((cid * NG + grp) * n_dst + lo + sid * rpt,
                                     rpt)])
                plsc.subcore_barrier()
            return carry0

        lax.fori_loop(0, NG * halves, pass_body, 0)

    return k


def _make_tgt_gather():
    @functools.partial(
        pl.kernel, mesh=_mesh, compiler_params=_sc_params,
        out_type=jax.ShapeDtypeStruct((NC * BS, 32), jnp.float32),
        scratch_types=[
            pltpu.VMEM((SUB,), jnp.int32),
            pltpu.VMEM((SUB,), jnp.int32),
            pltpu.VMEM((SUB, 32), jnp.float32),
            pltpu.SemaphoreType.DMA,
        ],
    )
    def k(parts_hbm, idx_hbm, out_hbm, idx_v, idx2_v, rows_v, sem):
        w = lax.axis_index("c") * NS + lax.axis_index("s")
        pltpu.sync_copy(idx_hbm.at[w], idx_v)
        pltpu.async_copy(parts_hbm.at[idx_v], rows_v, sem).wait()
        pltpu.sync_copy(rows_v, out_hbm.at[pl.ds(w * SUB, SUB)])
        for m in range(NSUB):
            v = plsc.load_gather(idx_v, [_iota16() + m * 16])
            plsc.store_scatter(idx2_v, [_iota16() + m * 16], v + N_ITEM)
        pltpu.async_copy(parts_hbm.at[idx2_v], rows_v, sem).wait()
        pltpu.sync_copy(rows_v, out_hbm.at[pl.ds(BS + w * SUB, SUB)])

    return k


def _tc_item_l1(rows_id, rows_rat, ts, W_item, b_item, W1bt, al1bt, W1hi, ar1hi):
    BN = 2000
    grid = N_ITEM // BN

    def body(id_ref, rat_ref, ts_ref, Wi_ref, bi_ref, Wbt_ref, albt_ref,
             Whi_ref, arhi_ref, f_ref, el_ref, er_ref, mx_ref):
        x = jnp.concatenate([id_ref[...], rat_ref[...], ts_ref[...]], axis=1)
        emb = x @ Wi_ref[...] + bi_ref[...]
        f = emb @ Wbt_ref[...]
        gmat = emb @ Whi_ref[...]
        els = []
        ers = []
        for h in range(H1):
            fh = f[:, 32 * h:32 * h + 32]
            f_ref[h] = fh
            els.append(jnp.sum(fh * albt_ref[h][None, :], axis=1, keepdims=True))
            ers.append(jnp.sum(gmat[:, 32 * h:32 * h + 32] * arhi_ref[h][None, :],
                               axis=1, keepdims=True))
        z = jnp.zeros((BN, 8), jnp.float32)
        el16 = jnp.concatenate(els + [z], axis=1)
        er16 = jnp.concatenate(ers + [z], axis=1)
        el_ref[...] = el16
        er_ref[...] = er16
        m = jnp.concatenate([jnp.max(el16[:, :8], axis=0),
                             jnp.max(er16[:, :8], axis=0)])[None, :]

        @pl.when(pl.program_id(0) == 0)
        def _():
            mx_ref[...] = jnp.full((1, 16), -1e30, jnp.float32)

        mx_ref[...] = jnp.maximum(mx_ref[...], m)

    return pl.pallas_call(
        body,
        grid=(grid,),
        in_specs=[
            pl.BlockSpec((BN, 32), lambda i: (i, 0)),
            pl.BlockSpec((BN, 32), lambda i: (i, 0)),
            pl.BlockSpec((BN, 32), lambda i: (i, 0)),
            pl.BlockSpec((96, 32), lambda i: (0, 0)),
            pl.BlockSpec((1, 32), lambda i: (0, 0)),
            pl.BlockSpec((32, 256), lambda i: (0, 0)),
            pl.BlockSpec((8, 32), lambda i: (0, 0)),
            pl.BlockSpec((32, 256), lambda i: (0, 0)),
            pl.BlockSpec((8, 32), lambda i: (0, 0)),
        ],
        out_specs=[
            pl.BlockSpec((8, BN, 32), lambda i: (0, i, 0)),
            pl.BlockSpec((BN, 16), lambda i: (i, 0)),
            pl.BlockSpec((BN, 16), lambda i: (i, 0)),
            pl.BlockSpec((1, 16), lambda i: (0, 0)),
        ],
        out_shape=[
            jax.ShapeDtypeStruct((8, N_ITEM, 32), jnp.float32),
            jax.ShapeDtypeStruct((N_ITEM, 16), jnp.float32),
            jax.ShapeDtypeStruct((N_ITEM, 16), jnp.float32),
            jax.ShapeDtypeStruct((1, 16), jnp.float32),
        ],
    )(rows_id, rows_rat, ts, W_item, b_item, W1bt, al1bt, W1hi, ar1hi)


def _tc_feat_l1(rows_feat, W_feat, b_feat, W1hi, al1hi, W1bt, ar1bt):
    BN = 2000
    grid = N_FEAT // BN

    def body(rf_ref, Wf_ref, bf_ref, Whi_ref, alhi_ref, Wbt_ref, arbt_ref,
             f_ref, el_ref, er_ref, mx_ref):
        emb = rf_ref[...] @ Wf_ref[...] + bf_ref[...]
        f = emb @ Whi_ref[...]
        gmat = emb @ Wbt_ref[...]
        els = []
        ers = []
        for h in range(H1):
            fh = f[:, 32 * h:32 * h + 32]
            f_ref[h] = fh
            els.append(jnp.sum(fh * alhi_ref[h][None, :], axis=1, keepdims=True))
            ers.append(jnp.sum(gmat[:, 32 * h:32 * h + 32] * arbt_ref[h][None, :],
                               axis=1, keepdims=True))
        z = jnp.zeros((BN, 8), jnp.float32)
        el16 = jnp.concatenate(els + [z], axis=1)
        er16 = jnp.concatenate(ers + [z], axis=1)
        el_ref[...] = el16
        er_ref[...] = er16
        m = jnp.concatenate([jnp.max(el16[:, :8], axis=0),
                             jnp.max(er16[:, :8], axis=0)])[None, :]

        @pl.when(pl.program_id(0) == 0)
        def _():
            mx_ref[...] = jnp.full((1, 16), -1e30, jnp.float32)

        mx_ref[...] = jnp.maximum(mx_ref[...], m)

    return pl.pallas_call(
        body,
        grid=(grid,),
        in_specs=[
            pl.BlockSpec((BN, 32), lambda i: (i, 0)),
            pl.BlockSpec((32, 32), lambda i: (0, 0)),
            pl.BlockSpec((1, 32), lambda i: (0, 0)),
            pl.BlockSpec((32, 256), lambda i: (0, 0)),
            pl.BlockSpec((8, 32), lambda i: (0, 0)),
            pl.BlockSpec((32, 256), lambda i: (0, 0)),
            pl.BlockSpec((8, 32), lambda i: (0, 0)),
        ],
        out_specs=[
            pl.BlockSpec((8, BN, 32), lambda i: (0, i, 0)),
            pl.BlockSpec((BN, 16), lambda i: (i, 0)),
            pl.BlockSpec((BN, 16), lambda i: (i, 0)),
            pl.BlockSpec((1, 16), lambda i: (0, 0)),
        ],
        out_shape=[
            jax.ShapeDtypeStruct((H1, N_FEAT, 32), jnp.float32),
            jax.ShapeDtypeStruct((N_FEAT, 16), jnp.float32),
            jax.ShapeDtypeStruct((N_FEAT, 16), jnp.float32),
            jax.ShapeDtypeStruct((1, 16), jnp.float32),
        ],
    )(rows_feat, W_feat, b_feat, W1hi, al1hi, W1bt, ar1bt)


def _tc_l2_prep(parts, b1, W2a, al2a, W2b, ar2b, n, NG, G):
    BN = 2000
    grid = n // BN
    W = G * 32

    def body(p_ref, b1_ref, Wa_ref, ala_ref, Wb_ref, arb_ref,
             f_ref, el_ref, er_ref, mx_ref):
        cols = []
        for h in range(H1):
            grp, jj = divmod(h, G)
            cols.append(p_ref[0, grp, :, 32 * jj:32 * jj + 32]
                        + p_ref[1, grp, :, 32 * jj:32 * jj + 32])
        hmat = jnp.concatenate(cols, axis=1) + b1_ref[...]
        f2 = hmat @ Wa_ref[...]
        g2 = hmat @ Wb_ref[...]
        f_ref[...] = f2
        el = jnp.sum(f2 * ala_ref[0][None, :], axis=1, keepdims=True)
        er = jnp.sum(g2 * arb_ref[0][None, :], axis=1, keepdims=True)
        z = jnp.zeros((BN, 15), jnp.float32)
        el16 = jnp.concatenate([el, z], axis=1)
        er16 = jnp.concatenate([er, z], axis=1)
        el_ref[...] = el16
        er_ref[...] = er16
        m = jnp.concatenate(
            [jnp.max(el16[:, :8], axis=0), jnp.max(er16[:, :8], axis=0)])[None, :]

        @pl.when(pl.program_id(0) == 0)
        def _():
            mx_ref[...] = jnp.full((1, 16), -1e30, jnp.float32)

        mx_ref[...] = jnp.maximum(mx_ref[...], m)

    return pl.pallas_call(
        body,
        grid=(grid,),
        in_specs=[
            pl.BlockSpec((2, NG, BN, W), lambda i: (0, 0, i, 0)),
            pl.BlockSpec((1, 256), lambda i: (0, 0)),
            pl.BlockSpec((256, 32), lambda i: (0, 0)),
            pl.BlockSpec((1, 32), lambda i: (0, 0)),
            pl.BlockSpec((256, 32), lambda i: (0, 0)),
            pl.BlockSpec((1, 32), lambda i: (0, 0)),
        ],
        out_specs=[
            pl.BlockSpec((BN, 32), lambda i: (i, 0)),
            pl.BlockSpec((BN, 16), lambda i: (i, 0)),
            pl.BlockSpec((BN, 16), lambda i: (i, 0)),
            pl.BlockSpec((1, 16), lambda i: (0, 0)),
        ],
        out_shape=[
            jax.ShapeDtypeStruct((n, 32), jnp.float32),
            jax.ShapeDtypeStruct((n, 16), jnp.float32),
            jax.ShapeDtypeStruct((n, 16), jnp.float32),
            jax.ShapeDtypeStruct((1, 16), jnp.float32),
        ],
    )(parts, b1, W2a, al2a, W2b, ar2b)


def _tc_final(rows_user, W_user, b_user, tgt_parts, b2hi):
    def body(ru_ref, Wu_ref, bu_ref, tp_ref, b2_ref, o_ref):
        u = ru_ref[...] @ Wu_ref[...] + bu_ref[...]
        t = tp_ref[0] + tp_ref[1] + b2_ref[...]
        num = jnp.sum(u * t, axis=1)
        nu = jnp.maximum(jnp.sqrt(jnp.sum(u * u, axis=1)), 1e-8)
        nt = jnp.maximum(jnp.sqrt(jnp.sum(t * t, axis=1)), 1e-8)
        o_ref[...] = jax.nn.sigmoid(num / (nu * nt))

    return pl.pallas_call(
        body,
        out_shape=jax.ShapeDtypeStruct((BS,), jnp.float32),
    )(rows_user, W_user, b_user, tgt_parts, b2hi)


def _pad_edges(x):
    return jnp.pad(x, (0, E_PAD - E)).reshape(E_PAD // SUB, SUB).astype(jnp.int32)


def _c16(m, nh):
    c = jax.nn.leaky_relu(m, 0.2)
    return jnp.concatenate([c[:nh], jnp.full((16 - nh,), 100.0, jnp.float32)])


def kernel(user_feat, item_id, item_rating, feat_id, is_target,
           edge_bt_src, edge_bt_dst, edge_hi_src, edge_hi_dst,
           ts_emb, emb_table, W_user, b_user, W_item, b_item, W_feat, b_feat,
           W1_bt, al1_bt, ar1_bt, b1_bt, W1_hi, al1_hi, ar1_hi, b1_hi,
           W2_bt, al2_bt, ar2_bt, b2_bt, W2_hi, al2_hi, ar2_hi, b2_hi):
    zeros16 = jnp.zeros((2500, 16), jnp.float32)

    n_all = N_ITEM + N_FEAT + BS + N_ITEM
    n_all_pad = ((n_all + NW * SUB - 1) // (NW * SUB)) * (NW * SUB)
    idx_all = jnp.concatenate([
        item_id.astype(jnp.int32), feat_id.astype(jnp.int32),
        user_feat[:, 0].astype(jnp.int32), item_rating.astype(jnp.int32),
        jnp.zeros((n_all_pad - n_all,), jnp.int32)]).reshape(n_all_pad // SUB, SUB)
    rows = _make_gather(n_all_pad, emb_table.shape[0])(emb_table, idx_all)
    rows_id = rows[:N_ITEM]
    rows_feat = rows[N_ITEM:N_ITEM + N_FEAT]
    rows_user = rows[N_ITEM + N_FEAT:N_ITEM + N_FEAT + BS]
    rows_rat = rows[N_ITEM + N_FEAT + BS:n_all]

    f1_bt, el1bt, er1hi, mx_i = _tc_item_l1(
        rows_id, rows_rat, ts_emb, W_item, b_item.reshape(1, 32),
        W1_bt, al1_bt, W1_hi, ar1_hi)
    f1_hi, el1hi, er1bt, mx_f = _tc_feat_l1(
        rows_feat, W_feat, b_feat.reshape(1, 32), W1_hi, al1_hi, W1_bt, ar1_bt)

    c_bt1 = _c16(mx_i[0, :8] + mx_f[0, 8:], 8)
    c_hi1 = _c16(mx_f[0, :8] + mx_i[0, 8:], 8)

    sbt = _pad_edges(edge_bt_src)
    dbt = _pad_edges(edge_bt_dst)
    shi = _pad_edges(edge_hi_src)
    dhi = _pad_edges(edge_hi_dst)

    zeros_a = jnp.zeros((N_ITEM // NS, 16), jnp.float32)
    den_bt = _make_kA(N_FEAT)(sbt, dbt, el1bt, er1bt, c_bt1,
                              jnp.zeros((N_FEAT // NS, 16), jnp.float32))
    den_hi = _make_kA(N_ITEM)(shi, dhi, el1hi, er1hi, c_hi1, zeros_a)
    alpha_bt = _make_kB()(sbt, dbt, el1bt, er1bt,
                          den_bt[:N_FEAT], den_bt[N_FEAT:], c_bt1)
    alpha_hi = _make_kB()(shi, dhi, el1hi, er1hi,
                          den_hi[:N_ITEM], den_hi[N_ITEM:], c_hi1)
    ebt = jnp.bitwise_or(jnp.left_shift(dbt, 16), sbt)
    ehi = jnp.bitwise_or(jnp.left_shift(dhi, 16), shi)
    bt_parts = _make_kD(N_ITEM, N_FEAT, H1, 1)(
        ebt, alpha_bt, f1_bt.reshape(H1 * N_ITEM, 32),
        jnp.zeros((N_FEAT // NS, 32), jnp.float32))
    hi_parts = _make_kD(N_FEAT, N_ITEM, H1, 1)(
        ehi, alpha_hi, f1_hi.reshape(H1 * N_FEAT, 32),
        jnp.zeros((N_ITEM // NS, 32), jnp.float32))

    f2_bt, el2bt, er2hi, mx2_i = _tc_l2_prep(
        hi_parts.reshape(2, H1, N_ITEM, 32), b1_hi.reshape(1, 256),
        W2_bt, al2_bt, W2_hi, ar2_hi, N_ITEM, H1, 1)
    f2_hi, el2hi, er2bt, mx2_f = _tc_l2_prep(
        bt_parts.reshape(2, H1, N_FEAT, 32), b1_bt.reshape(1, 256),
        W2_hi, al2_hi, W2_bt, ar2_bt, N_FEAT, H1, 1)

    c_hi2 = _c16(mx2_f[0, :8] + mx2_i[0, 8:], 1)

    den2_hi = _make_kA(N_ITEM)(shi, dhi, el2hi, er2hi, c_hi2, zeros_a)
    alpha2_hi = _make_kB()(shi, dhi, el2hi, er2hi,
                           den2_hi[:N_ITEM], den2_hi[N_ITEM:], c_hi2)
    hi2_parts = _make_kD(N_FEAT, N_ITEM, 1, 1)(
        ehi, alpha2_hi, f2_hi,
        jnp.zeros((N_ITEM // NS, 32), jnp.float32))

    idx_tgt = jnp.nonzero(is_target, size=BS, fill_value=0)[0].astype(jnp.int32)
    tgt_parts = _make_tgt_gather()(hi2_parts, idx_tgt.reshape(NW, SUB))
    return _tc_final(rows_user, W_user, b_user.reshape(1, 32),
                     tgt_parts.reshape(2, BS, 32), b2_hi.reshape(1, 32))
